# Initial kernel scaffold; baseline (speedup 1.0000x reference)
import functools

import jax
import jax.numpy as jnp
from jax import lax
from jax.experimental import pallas as pl
from jax.experimental.pallas import tpu as pltpu
from jax.experimental.pallas import tpu_sc as plsc

N = 10000
NP = 10240
E = 320000
G = 8
NC = 2
NS = 16
NW = NC * NS
EW = E // NW
B = 400
NB = EW // B
RPS = NP // NS
F = 64

_mesh = plsc.VectorSubcoreMesh(core_axis_name="c", subcore_axis_name="s")


def _deg_body(src_hbm, dst_hbm, w_hbm, zeros_hbm, deg_out,
              si, di, wv, ev, acc):
    c = lax.axis_index("c")
    s = lax.axis_index("s")
    wid = c * NS + s
    rowbase = s * RPS
    pltpu.sync_copy(zeros_hbm.at[pl.ds(rowbase, RPS)],
                    acc.at[pl.ds(rowbase, RPS)])
    plsc.subcore_barrier()
    ebase = wid * EW

    @pl.loop(0, NB)
    def _(blk):
        e0 = ebase + blk * B
        pltpu.sync_copy(src_hbm.at[pl.ds(e0, B)], si)
        pltpu.sync_copy(dst_hbm.at[pl.ds(e0, B)], di)
        pltpu.sync_copy(w_hbm.at[pl.ds(e0, B)], wv)
        for g in range(B // 16):
            sl = pl.ds(g * 16, 16)
            s16 = si[sl]
            d16 = di[sl]
            w16 = wv[sl]
            ev[sl] = jnp.where(s16 != d16, w16, 0.0)
        pltpu.sync_copy(ev, acc.at[si], add=True)

    plsc.subcore_barrier()
    pltpu.sync_copy(acc.at[pl.ds(rowbase, RPS)],
                    deg_out.at[c, pl.ds(rowbase, RPS)])


def _deg_kernel(src, dst, w, zeros1):
    return pl.kernel(
        _deg_body,
        out_type=jax.ShapeDtypeStruct((NC, NP), jnp.float32),
        mesh=_mesh,
        scratch_types=[
            pltpu.VMEM((B,), jnp.int32),
            pltpu.VMEM((B,), jnp.int32),
            pltpu.VMEM((B,), jnp.float32),
            pltpu.VMEM((B,), jnp.float32),
            pltpu.VMEM_SHARED((NP,), jnp.float32),
        ],
    )(src, dst, w, zeros1)


def _spmm_body(compute_norm, z_hbm, src_hbm, dst_hbm, nrm_hbm, dis_hbm,
               zeros_hbm, out, *rest):
    if compute_norm:
        norm_out, si, di, nv, rows, disv, stage, acc = rest
    else:
        si, di, nv, rows, disv, stage, acc = rest
    c = lax.axis_index("c")
    s = lax.axis_index("s")
    wid = c * NS + s
    rowbase = s * RPS
    pltpu.sync_copy(zeros_hbm.at[pl.ds(rowbase, RPS)],
                    acc.at[pl.ds(rowbase, RPS)])
    pltpu.sync_copy(dis_hbm, disv)
    plsc.subcore_barrier()
    ebase = wid * EW

    @pl.loop(0, NB)
    def _(blk):
        e0 = ebase + blk * B
        pltpu.sync_copy(src_hbm.at[pl.ds(e0, B)], si)
        pltpu.sync_copy(dst_hbm.at[pl.ds(e0, B)], di)
        pltpu.sync_copy(nrm_hbm.at[pl.ds(e0, B)], nv)
        if compute_norm:
            for g in range(B // 16):
                sl = pl.ds(g * 16, 16)
                s16 = si[sl]
                d16 = di[sl]
                w16 = nv[sl]
                g16 = plsc.load_gather(disv, [s16])
                nv[sl] = jnp.where(s16 != d16, -w16 * g16, 0.0)
            pltpu.sync_copy(nv, norm_out.at[pl.ds(e0, B)])
        pltpu.sync_copy(z_hbm.at[si], rows)

        @pl.loop(0, B)
        def _(i):
            n16 = plsc.load_gather(nv, [jnp.zeros((16,), jnp.int32) + i])
            row = rows.at[i]
            for ch in range(F // 16):
                cs = pl.ds(ch * 16, 16)
                row[cs] = row[cs] * n16

        pltpu.sync_copy(rows, acc.at[di], add=True)

    plsc.subcore_barrier()
    pltpu.sync_copy(acc.at[pl.ds(rowbase, RPS)], stage)

    @pl.loop(0, RPS)
    def _(r):
        d16 = plsc.load_gather(disv, [jnp.zeros((16,), jnp.int32)
                                      + (rowbase + r)])
        row = stage.at[r]
        for ch in range(F // 16):
            cs = pl.ds(ch * 16, 16)
            row[cs] = row[cs] * d16

    pltpu.sync_copy(stage, out.at[c, pl.ds(rowbase, RPS)])


def _make_spmm(compute_norm):
    out_type = [jax.ShapeDtypeStruct((NC, NP, F), jnp.float32)]
    if compute_norm:
        out_type.append(jax.ShapeDtypeStruct((E,), jnp.float32))
    return pl.kernel(
        functools.partial(_spmm_body, compute_norm),
        out_type=out_type,
        mesh=_mesh,
        scratch_types=[
            pltpu.VMEM((B,), jnp.int32),
            pltpu.VMEM((B,), jnp.int32),
            pltpu.VMEM((B,), jnp.float32),
            pltpu.VMEM((B, F), jnp.float32),
            pltpu.VMEM((NP,), jnp.float32),
            pltpu.VMEM((RPS, F), jnp.float32),
            pltpu.VMEM_SHARED((NP, F), jnp.float32),
        ],
    )


_spmm_first = _make_spmm(True)
_spmm_next = _make_spmm(False)


def _tc1_body(x_ref, w_ref, b_ref, deg_ref, dis_out, a1_out, y1_out, y2_out):
    deg = deg_ref[0] + deg_ref[1]
    dis_out[...] = jnp.where(deg > 0, lax.rsqrt(jnp.where(deg > 0, deg, 1.0)),
                             0.0)
    y = jnp.dot(x_ref[...], w_ref[...], preferred_element_type=jnp.float32)
    y2 = y[:, 2 * F:]
    a1_out[...] = y[:, :F] - y2 + b_ref[...]
    y1_out[...] = y[:, F:2 * F]
    y2_out[...] = y2


def _tc1(xp, w1cat, b1, degp):
    return pl.pallas_call(
        _tc1_body,
        out_shape=[
            jax.ShapeDtypeStruct((NP // 128, 128), jnp.float32),
            jax.ShapeDtypeStruct((NP, F), jnp.float32),
            jax.ShapeDtypeStruct((NP, F), jnp.float32),
            jax.ShapeDtypeStruct((NP, F), jnp.float32),
        ],
    )(xp, w1cat, b1, degp)


def _tc3_body(y1_ref, u_ref, v_out):
    v_out[...] = y1_ref[...] + 2.0 * (u_ref[0] + u_ref[1])


def _tc3(y1, u):
    return pl.pallas_call(
        _tc3_body,
        out_shape=jax.ShapeDtypeStruct((NP, F), jnp.float32),
    )(y1, u)


def _tc4_body(a1_ref, sp_ref, w_ref, b_ref, h_out, c0_out):
    h = jnp.maximum(a1_ref[...] + sp_ref[0] + sp_ref[1], 0.0)
    h_out[...] = h
    c0_out[...] = jnp.dot(h, w_ref[...],
                          preferred_element_type=jnp.float32) + b_ref[...]


def _tc4(a1, sp, w20, b2):
    return pl.pallas_call(
        _tc4_body,
        out_shape=[
            jax.ShapeDtypeStruct((NP, F), jnp.float32),
            jax.ShapeDtypeStruct((NP, F), jnp.float32),
        ],
    )(a1, sp, w20, b2)


def _tc5_body(c0_ref, t_ref, w_ref, tx_out, c01_out):
    tx = t_ref[0] + t_ref[1]
    tx_out[...] = tx
    c01_out[...] = c0_ref[...] + jnp.dot(tx, w_ref[...],
                                         preferred_element_type=jnp.float32)


def _tc5(c0, t, w):
    fo = w.shape[-1]
    return pl.pallas_call(
        _tc5_body,
        out_shape=[
            jax.ShapeDtypeStruct((NP, F), jnp.float32),
            jax.ShapeDtypeStruct((NP, fo), jnp.float32),
        ],
    )(c0, t, w)


def _tc6_body(c01_ref, t_ref, h_ref, w22_ref, w30_ref, b3_ref, h2_out,
              c0b_out):
    tx2 = 2.0 * (t_ref[0] + t_ref[1]) - h_ref[...]
    h2 = jnp.maximum(
        c01_ref[...] + jnp.dot(tx2, w22_ref[...],
                               preferred_element_type=jnp.float32), 0.0)
    h2_out[...] = h2
    c0b_out[...] = jnp.dot(h2, w30_ref[...],
                           preferred_element_type=jnp.float32) + b3_ref[...]


def _tc6(c01, t2, h, w22, w30, b3):
    return pl.pallas_call(
        _tc6_body,
        out_shape=[
            jax.ShapeDtypeStruct((NP, F), jnp.float32),
            jax.ShapeDtypeStruct((NP, 128), jnp.float32),
        ],
    )(c01, t2, h, w22, w30, b3)


def _tc8_body(c01_ref, t_ref, h2_ref, w32_ref, batch_ref, wl_ref, bl_ref,
              out_ref):
    tx2 = 2.0 * (t_ref[0] + t_ref[1]) - h2_ref[...]
    h3 = jnp.maximum(
        c01_ref[...] + jnp.dot(tx2, w32_ref[...],
                               preferred_element_type=jnp.float32), 0.0)
    gids = lax.broadcasted_iota(jnp.int32, (1, G), 1)
    oh = (batch_ref[...] == gids).astype(jnp.float32)
    seg = lax.dot_general(oh, h3, (((0,), (0,)), ((), ())),
                          preferred_element_type=jnp.float32)
    cnt = jnp.sum(oh, axis=0)
    pooled = seg / jnp.maximum(cnt, 1.0)[:, None]
    out_ref[...] = jnp.dot(pooled, wl_ref[...],
                           preferred_element_type=jnp.float32) + bl_ref[...]


def _tc8(c01b, u2, h2, w32, batchp, wl, bl):
    return pl.pallas_call(
        _tc8_body,
        out_shape=jax.ShapeDtypeStruct((G, wl.shape[-1]), jnp.float32),
    )(c01b, u2, h2, w32, batchp, wl, bl)


def kernel(x, edge_index, edge_attr, batch, W1, b1, W2, b2, W3, b3, Wl, bl):
    src = edge_index[0].astype(jnp.int32)
    dst = edge_index[1].astype(jnp.int32)
    w = edge_attr.astype(jnp.float32)
    xp = jnp.pad(x, ((0, NP - N), (0, 0)))
    batchp = jnp.pad(batch.astype(jnp.int32), (0, NP - N),
                     constant_values=G).reshape(NP, 1)
    zeros1 = jnp.zeros((NP,), jnp.float32)
    zeros2 = jnp.zeros((NP, F), jnp.float32)
    w1cat = jnp.concatenate([W1[0], W1[1], W1[2]], axis=1)

    degp = _deg_kernel(src, dst, w, zeros1)
    dis80, a1, y1, y2 = _tc1(xp, w1cat, b1.reshape(1, F),
                             degp.reshape(NC, NP // 128, 128))
    dis = dis80.reshape(NP)

    u, norm = _spmm_first(y2, src, dst, w, dis, zeros2)
    v = _tc3(y1, u)
    sp = _spmm_next(v, src, dst, norm, dis, zeros2)
    h, c0 = _tc4(a1, sp, W2[0], b2.reshape(1, F))

    t1 = _spmm_next(h, src, dst, norm, dis, zeros2)
    tx1, c01 = _tc5(c0, t1, W2[1])
    t2 = _spmm_next(tx1, src, dst, norm, dis, zeros2)
    h2, c0b = _tc6(c01, t2, h, W2[2], W3[0], b3.reshape(1, 128))

    u1 = _spmm_next(h2, src, dst, norm, dis, zeros2)
    u1s, c01b = _tc5(c0b, u1, W3[1])
    u2 = _spmm_next(u1s, src, dst, norm, dis, zeros2)
    return _tc8(c01b, u2, h2, W3[2], batchp, Wl, bl.reshape(1, Wl.shape[-1]))


# speedup vs baseline: 8.9844x; 8.9844x over previous
import dataclasses
import functools

import jax
import jax.numpy as jnp
from jax import lax
from jax.experimental import pallas as pl
from jax.experimental.pallas import tpu as pltpu
from jax.experimental.pallas import tpu_sc as plsc

N = 10000
NP = 10240
E = 320000
G = 8
NC = 2
NS = 16
NW = NC * NS
EW = E // NW
B = 400
NB = EW // B
RPS = NP // NS
F = 64

_mesh = plsc.VectorSubcoreMesh(core_axis_name="c", subcore_axis_name="s")

_sc_params = pltpu.CompilerParams()
if "needs_layout_passes" in pltpu.CompilerParams.__dataclass_fields__:
    _sc_params = dataclasses.replace(_sc_params, needs_layout_passes=False)
if "use_tc_tiling_on_sc" in pltpu.CompilerParams.__dataclass_fields__:
    _sc_params = dataclasses.replace(_sc_params, use_tc_tiling_on_sc=False)


def _deg_body(src_hbm, dst_hbm, w_hbm, zeros_hbm, deg_out,
              si, di, wv, ev, acc):
    c = lax.axis_index("c")
    s = lax.axis_index("s")
    wid = c * NS + s
    rowbase = s * RPS
    pltpu.sync_copy(zeros_hbm.at[pl.ds(rowbase, RPS)],
                    acc.at[pl.ds(rowbase, RPS)])
    plsc.subcore_barrier()
    ebase = wid * EW

    @pl.loop(0, NB)
    def _(blk):
        e0 = ebase + blk * B
        pltpu.sync_copy(src_hbm.at[pl.ds(e0, B)], si)
        pltpu.sync_copy(dst_hbm.at[pl.ds(e0, B)], di)
        pltpu.sync_copy(w_hbm.at[pl.ds(e0, B)], wv)
        for g in range(B // 16):
            sl = pl.ds(g * 16, 16)
            s16 = si[sl]
            d16 = di[sl]
            w16 = wv[sl]
            ev[sl] = jnp.where(s16 != d16, w16, 0.0)
        pltpu.sync_copy(ev, acc.at[si], add=True)

    plsc.subcore_barrier()
    pltpu.sync_copy(acc.at[pl.ds(rowbase, RPS)],
                    deg_out.at[c, pl.ds(rowbase, RPS)])


def _deg_kernel(src, dst, w, zeros1):
    return pl.kernel(
        _deg_body,
        out_type=jax.ShapeDtypeStruct((NC, NP), jnp.float32),
        mesh=_mesh,
        scratch_types=[
            pltpu.VMEM((B,), jnp.int32),
            pltpu.VMEM((B,), jnp.int32),
            pltpu.VMEM((B,), jnp.float32),
            pltpu.VMEM((B,), jnp.float32),
            pltpu.VMEM_SHARED((NP,), jnp.float32),
        ],
        compiler_params=_sc_params,
    )(src, dst, w, zeros1)


def _spmm_body(compute_norm, z_hbm, src_hbm, dst_hbm, nrm_hbm, dis_hbm,
               zeros_hbm, out, *rest):
    if compute_norm:
        norm_out, si, di, nv, rows, disv, stage, acc = rest
    else:
        si, di, nv, rows, disv, stage, acc = rest
    c = lax.axis_index("c")
    s = lax.axis_index("s")
    wid = c * NS + s
    rowbase = s * RPS
    pltpu.sync_copy(zeros_hbm.at[pl.ds(rowbase, RPS)],
                    acc.at[pl.ds(rowbase, RPS)])
    pltpu.sync_copy(dis_hbm, disv)
    plsc.subcore_barrier()
    ebase = wid * EW

    @pl.loop(0, NB)
    def _(blk):
        e0 = ebase + blk * B
        pltpu.sync_copy(src_hbm.at[pl.ds(e0, B)], si)
        pltpu.sync_copy(dst_hbm.at[pl.ds(e0, B)], di)
        pltpu.sync_copy(nrm_hbm.at[pl.ds(e0, B)], nv)
        if compute_norm:
            for g in range(B // 16):
                sl = pl.ds(g * 16, 16)
                s16 = si[sl]
                d16 = di[sl]
                w16 = nv[sl]
                g16 = plsc.load_gather(disv, [s16])
                nv[sl] = jnp.where(s16 != d16, -w16 * g16, 0.0)
            pltpu.sync_copy(nv, norm_out.at[pl.ds(e0, B)])
        pltpu.sync_copy(z_hbm.at[si], rows)

        @pl.loop(0, B)
        def _(i):
            n16 = plsc.load_gather(nv, [jnp.zeros((16,), jnp.int32) + i])
            row = rows.at[i]
            for ch in range(F // 16):
                cs = pl.ds(ch * 16, 16)
                row[cs] = row[cs] * n16

        pltpu.sync_copy(rows, acc.at[di], add=True)

    plsc.subcore_barrier()
    pltpu.sync_copy(acc.at[pl.ds(rowbase, RPS)], stage)

    @pl.loop(0, RPS)
    def _(r):
        d16 = plsc.load_gather(disv, [jnp.zeros((16,), jnp.int32)
                                      + (rowbase + r)])
        row = stage.at[r]
        for ch in range(F // 16):
            cs = pl.ds(ch * 16, 16)
            row[cs] = row[cs] * d16

    pltpu.sync_copy(stage, out.at[c, pl.ds(rowbase, RPS)])


def _make_spmm(compute_norm):
    out_type = jax.ShapeDtypeStruct((NC, NP, F), jnp.float32)
    if compute_norm:
        out_type = [out_type, jax.ShapeDtypeStruct((E,), jnp.float32)]
    return pl.kernel(
        functools.partial(_spmm_body, compute_norm),
        out_type=out_type,
        mesh=_mesh,
        scratch_types=[
            pltpu.VMEM((B,), jnp.int32),
            pltpu.VMEM((B,), jnp.int32),
            pltpu.VMEM((B,), jnp.float32),
            pltpu.VMEM((B, F), jnp.float32),
            pltpu.VMEM((NP,), jnp.float32),
            pltpu.VMEM((RPS, F), jnp.float32),
            pltpu.VMEM_SHARED((NP, F), jnp.float32),
        ],
        compiler_params=_sc_params,
    )


_spmm_first = _make_spmm(True)
_spmm_next = _make_spmm(False)


def _tc1_body(x_ref, w_ref, b_ref, deg_ref, dis_out, a1_out, y1_out, y2_out):
    deg = deg_ref[0] + deg_ref[1]
    dis_out[...] = jnp.where(deg > 0, lax.rsqrt(jnp.where(deg > 0, deg, 1.0)),
                             0.0)
    y = jnp.dot(x_ref[...], w_ref[...], preferred_element_type=jnp.float32)
    y2 = y[:, 2 * F:]
    a1_out[...] = y[:, :F] - y2 + b_ref[...]
    y1_out[...] = y[:, F:2 * F]
    y2_out[...] = y2


def _tc1(xp, w1cat, b1, degp):
    return pl.pallas_call(
        _tc1_body,
        out_shape=[
            jax.ShapeDtypeStruct((NP // 128, 128), jnp.float32),
            jax.ShapeDtypeStruct((NP, F), jnp.float32),
            jax.ShapeDtypeStruct((NP, F), jnp.float32),
            jax.ShapeDtypeStruct((NP, F), jnp.float32),
        ],
    )(xp, w1cat, b1, degp)


def _tc3_body(y1_ref, u_ref, v_out):
    v_out[...] = y1_ref[...] + 2.0 * (u_ref[0] + u_ref[1])


def _tc3(y1, u):
    return pl.pallas_call(
        _tc3_body,
        out_shape=jax.ShapeDtypeStruct((NP, F), jnp.float32),
    )(y1, u)


def _tc4_body(a1_ref, sp_ref, w_ref, b_ref, h_out, c0_out):
    h = jnp.maximum(a1_ref[...] + sp_ref[0] + sp_ref[1], 0.0)
    h_out[...] = h
    c0_out[...] = jnp.dot(h, w_ref[...],
                          preferred_element_type=jnp.float32) + b_ref[...]


def _tc4(a1, sp, w20, b2):
    return pl.pallas_call(
        _tc4_body,
        out_shape=[
            jax.ShapeDtypeStruct((NP, F), jnp.float32),
            jax.ShapeDtypeStruct((NP, F), jnp.float32),
        ],
    )(a1, sp, w20, b2)


def _tc5_body(c0_ref, t_ref, w_ref, tx_out, c01_out):
    tx = t_ref[0] + t_ref[1]
    tx_out[...] = tx
    c01_out[...] = c0_ref[...] + jnp.dot(tx, w_ref[...],
                                         preferred_element_type=jnp.float32)


def _tc5(c0, t, w):
    fo = w.shape[-1]
    return pl.pallas_call(
        _tc5_body,
        out_shape=[
            jax.ShapeDtypeStruct((NP, F), jnp.float32),
            jax.ShapeDtypeStruct((NP, fo), jnp.float32),
        ],
    )(c0, t, w)


def _tc6_body(c01_ref, t_ref, h_ref, w22_ref, w30_ref, b3_ref, h2_out,
              c0b_out):
    tx2 = 2.0 * (t_ref[0] + t_ref[1]) - h_ref[...]
    h2 = jnp.maximum(
        c01_ref[...] + jnp.dot(tx2, w22_ref[...],
                               preferred_element_type=jnp.float32), 0.0)
    h2_out[...] = h2
    c0b_out[...] = jnp.dot(h2, w30_ref[...],
                           preferred_element_type=jnp.float32) + b3_ref[...]


def _tc6(c01, t2, h, w22, w30, b3):
    return pl.pallas_call(
        _tc6_body,
        out_shape=[
            jax.ShapeDtypeStruct((NP, F), jnp.float32),
            jax.ShapeDtypeStruct((NP, 128), jnp.float32),
        ],
    )(c01, t2, h, w22, w30, b3)


def _tc8_body(c01_ref, t_ref, h2_ref, w32_ref, batch_ref, wl_ref, bl_ref,
              out_ref):
    tx2 = 2.0 * (t_ref[0] + t_ref[1]) - h2_ref[...]
    h3 = jnp.maximum(
        c01_ref[...] + jnp.dot(tx2, w32_ref[...],
                               preferred_element_type=jnp.float32), 0.0)
    gids = lax.broadcasted_iota(jnp.int32, (1, G), 1)
    oh = (batch_ref[...] == gids).astype(jnp.float32)
    seg = lax.dot_general(oh, h3, (((0,), (0,)), ((), ())),
                          preferred_element_type=jnp.float32)
    cnt = jnp.sum(oh, axis=0)
    pooled = seg / jnp.maximum(cnt, 1.0)[:, None]
    out_ref[...] = jnp.dot(pooled, wl_ref[...],
                           preferred_element_type=jnp.float32) + bl_ref[...]


def _tc8(c01b, u2, h2, w32, batchp, wl, bl):
    return pl.pallas_call(
        _tc8_body,
        out_shape=jax.ShapeDtypeStruct((G, wl.shape[-1]), jnp.float32),
    )(c01b, u2, h2, w32, batchp, wl, bl)


def kernel(x, edge_index, edge_attr, batch, W1, b1, W2, b2, W3, b3, Wl, bl):
    src = edge_index[0].astype(jnp.int32)
    dst = edge_index[1].astype(jnp.int32)
    w = edge_attr.astype(jnp.float32)
    xp = jnp.pad(x, ((0, NP - N), (0, 0)))
    batchp = jnp.pad(batch.astype(jnp.int32), (0, NP - N),
                     constant_values=G).reshape(NP, 1)
    zeros1 = jnp.zeros((NP,), jnp.float32)
    zeros2 = jnp.zeros((NP, F), jnp.float32)
    w1cat = jnp.concatenate([W1[0], W1[1], W1[2]], axis=1)

    degp = _deg_kernel(src, dst, w, zeros1)
    dis80, a1, y1, y2 = _tc1(xp, w1cat, b1.reshape(1, F),
                             degp.reshape(NC, NP // 128, 128))
    dis = dis80.reshape(NP)

    u, norm = _spmm_first(y2, src, dst, w, dis, zeros2)
    v = _tc3(y1, u)
    sp = _spmm_next(v, src, dst, norm, dis, zeros2)
    h, c0 = _tc4(a1, sp, W2[0], b2.reshape(1, F))

    t1 = _spmm_next(h, src, dst, norm, dis, zeros2)
    tx1, c01 = _tc5(c0, t1, W2[1])
    t2 = _spmm_next(tx1, src, dst, norm, dis, zeros2)
    h2, c0b = _tc6(c01, t2, h, W2[2], W3[0], b3.reshape(1, 128))

    u1 = _spmm_next(h2, src, dst, norm, dis, zeros2)
    u1s, c01b = _tc5(c0b, u1, W3[1])
    u2 = _spmm_next(u1s, src, dst, norm, dis, zeros2)
    return _tc8(c01b, u2, h2, W3[2], batchp, Wl, bl.reshape(1, Wl.shape[-1]))


# speedup vs baseline: 10.7415x; 1.1956x over previous
import dataclasses
import functools

import jax
import jax.numpy as jnp
from jax import lax
from jax.experimental import pallas as pl
from jax.experimental.pallas import tpu as pltpu
from jax.experimental.pallas import tpu_sc as plsc

N = 10000
NP = 10240
E = 320000
G = 8
NC = 2
NS = 16
NW = NC * NS
EW = E // NW
B = 400
NB = EW // B
RPS = NP // NS
F = 64

_mesh = plsc.VectorSubcoreMesh(core_axis_name="c", subcore_axis_name="s")

_sc_params = pltpu.CompilerParams()
if "needs_layout_passes" in pltpu.CompilerParams.__dataclass_fields__:
    _sc_params = dataclasses.replace(_sc_params, needs_layout_passes=False)
if "use_tc_tiling_on_sc" in pltpu.CompilerParams.__dataclass_fields__:
    _sc_params = dataclasses.replace(_sc_params, use_tc_tiling_on_sc=False)


def _deg_body(src_hbm, dst_hbm, w_hbm, zeros_hbm, deg_out, wbar_out,
              sbig, dbig, wbig, si, acc):
    c = lax.axis_index("c")
    s = lax.axis_index("s")
    wid = c * NS + s
    rowbase = s * RPS
    ebase = wid * EW
    pltpu.sync_copy(zeros_hbm.at[pl.ds(rowbase, RPS)],
                    acc.at[pl.ds(rowbase, RPS)])
    pltpu.sync_copy(src_hbm.at[pl.ds(ebase, EW)], sbig)
    pltpu.sync_copy(dst_hbm.at[pl.ds(ebase, EW)], dbig)
    pltpu.sync_copy(w_hbm.at[pl.ds(ebase, EW)], wbig)
    plsc.subcore_barrier()

    @pl.loop(0, EW // 16, unroll=8)
    def _(g):
        sl = pl.ds(g * 16, 16)
        wbig[sl] = jnp.where(sbig[sl] != dbig[sl], wbig[sl], 0.0)

    pltpu.sync_copy(wbig, wbar_out.at[pl.ds(ebase, EW)])

    @pl.loop(0, NB)
    def _(blk):
        e0 = blk * B
        pltpu.sync_copy(src_hbm.at[pl.ds(ebase + e0, B)], si)
        pltpu.sync_copy(wbig.at[pl.ds(e0, B)], acc.at[si], add=True)

    plsc.subcore_barrier()
    pltpu.sync_copy(acc.at[pl.ds(rowbase, RPS)],
                    deg_out.at[c, pl.ds(rowbase, RPS)])


def _deg_kernel(src, dst, w, zeros1):
    return pl.kernel(
        _deg_body,
        out_type=[jax.ShapeDtypeStruct((NC, NP), jnp.float32),
                  jax.ShapeDtypeStruct((E,), jnp.float32)],
        mesh=_mesh,
        scratch_types=[
            pltpu.VMEM((EW,), jnp.int32),
            pltpu.VMEM((EW,), jnp.int32),
            pltpu.VMEM((EW,), jnp.float32),
            pltpu.VMEM((B,), jnp.int32),
            pltpu.VMEM_SHARED((NP,), jnp.float32),
        ],
        compiler_params=_sc_params,
    )(src, dst, w, zeros1)


def _spmm_body(compute_norm, z_hbm, src_hbm, dst_hbm, nrm_hbm, *rest):
    if compute_norm:
        (dis_hbm, zeros_hbm, out, norm_out,
         sbig, nbig, di, rows, disv, acc) = rest
    else:
        (zeros_hbm, out,
         sbig, nbig, di, rows, acc) = rest
    c = lax.axis_index("c")
    s = lax.axis_index("s")
    wid = c * NS + s
    rowbase = s * RPS
    ebase = wid * EW
    pltpu.sync_copy(zeros_hbm.at[pl.ds(rowbase, RPS)],
                    acc.at[pl.ds(rowbase, RPS)])
    pltpu.sync_copy(src_hbm.at[pl.ds(ebase, EW)], sbig)
    pltpu.sync_copy(nrm_hbm.at[pl.ds(ebase, EW)], nbig)
    if compute_norm:
        pltpu.sync_copy(dis_hbm, disv)
    plsc.subcore_barrier()

    if compute_norm:
        @pl.loop(0, EW // 16, unroll=8)
        def _(g):
            sl = pl.ds(g * 16, 16)
            nbig[sl] = -nbig[sl] * plsc.load_gather(disv, [sbig[sl]])

        pltpu.sync_copy(nbig, norm_out.at[pl.ds(ebase, EW)])

    @pl.loop(0, NB)
    def _(blk):
        e0 = blk * B
        pltpu.sync_copy(dst_hbm.at[pl.ds(ebase + e0, B)], di)
        pltpu.sync_copy(z_hbm.at[sbig.at[pl.ds(e0, B)]], rows)

        @pl.loop(0, B, unroll=8)
        def _(i):
            n16 = plsc.load_gather(
                nbig, [jnp.zeros((16,), jnp.int32) + (e0 + i)])
            row = rows.at[i]
            for ch in range(F // 16):
                cs = pl.ds(ch * 16, 16)
                row[cs] = row[cs] * n16

        pltpu.sync_copy(rows, acc.at[di], add=True)

    plsc.subcore_barrier()
    pltpu.sync_copy(acc.at[pl.ds(rowbase, RPS)],
                    out.at[c, pl.ds(rowbase, RPS)])


def _make_spmm(compute_norm):
    if compute_norm:
        out_type = [jax.ShapeDtypeStruct((NC, NP, F), jnp.float32),
                    jax.ShapeDtypeStruct((E,), jnp.float32)]
    else:
        out_type = jax.ShapeDtypeStruct((NC, NP, F), jnp.float32)
    scratch = [
        pltpu.VMEM((EW,), jnp.int32),
        pltpu.VMEM((EW,), jnp.float32),
        pltpu.VMEM((B,), jnp.int32),
        pltpu.VMEM((B, F), jnp.float32),
    ]
    if compute_norm:
        scratch.append(pltpu.VMEM((NP,), jnp.float32))
    scratch.append(pltpu.VMEM_SHARED((NP, F), jnp.float32))
    return pl.kernel(
        functools.partial(_spmm_body, compute_norm),
        out_type=out_type,
        mesh=_mesh,
        scratch_types=scratch,
        compiler_params=_sc_params,
    )


_spmm_first = _make_spmm(True)
_spmm_next = _make_spmm(False)


def _tc1_body(x_ref, w_ref, b_ref, deg_ref, dis_out, a1_out, y1_out, y2_out):
    deg = deg_ref[0] + deg_ref[1]
    dis_out[...] = jnp.where(deg > 0, lax.rsqrt(jnp.where(deg > 0, deg, 1.0)),
                             0.0)
    y = jnp.dot(x_ref[...], w_ref[...], preferred_element_type=jnp.float32)
    y2 = y[:, 2 * F:]
    a1_out[...] = y[:, :F] - y2 + b_ref[...]
    y1_out[...] = y[:, F:2 * F]
    y2_out[...] = y2


def _tc1(xp, w1cat, b1, degp):
    return pl.pallas_call(
        _tc1_body,
        out_shape=[
            jax.ShapeDtypeStruct((NP // 128, 128), jnp.float32),
            jax.ShapeDtypeStruct((NP, F), jnp.float32),
            jax.ShapeDtypeStruct((NP, F), jnp.float32),
            jax.ShapeDtypeStruct((NP, F), jnp.float32),
        ],
    )(xp, w1cat, b1, degp)


def _tc3_body(y1_ref, u_ref, dis_ref, v_out):
    v_out[...] = y1_ref[...] + (2.0 * dis_ref[...]) * (u_ref[0] + u_ref[1])


def _tc3(y1, u, dis_col):
    return pl.pallas_call(
        _tc3_body,
        out_shape=jax.ShapeDtypeStruct((NP, F), jnp.float32),
    )(y1, u, dis_col)


def _tc4_body(a1_ref, sp_ref, dis_ref, w_ref, b_ref, h_out, c0_out):
    h = jnp.maximum(a1_ref[...] + dis_ref[...] * (sp_ref[0] + sp_ref[1]), 0.0)
    h_out[...] = h
    c0_out[...] = jnp.dot(h, w_ref[...],
                          preferred_element_type=jnp.float32) + b_ref[...]


def _tc4(a1, sp, dis_col, w20, b2):
    return pl.pallas_call(
        _tc4_body,
        out_shape=[
            jax.ShapeDtypeStruct((NP, F), jnp.float32),
            jax.ShapeDtypeStruct((NP, F), jnp.float32),
        ],
    )(a1, sp, dis_col, w20, b2)


def _tc5_body(c0_ref, t_ref, dis_ref, w_ref, tx_out, c01_out):
    tx = dis_ref[...] * (t_ref[0] + t_ref[1])
    tx_out[...] = tx
    c01_out[...] = c0_ref[...] + jnp.dot(tx, w_ref[...],
                                         preferred_element_type=jnp.float32)


def _tc5(c0, t, dis_col, w):
    fo = w.shape[-1]
    return pl.pallas_call(
        _tc5_body,
        out_shape=[
            jax.ShapeDtypeStruct((NP, F), jnp.float32),
            jax.ShapeDtypeStruct((NP, fo), jnp.float32),
        ],
    )(c0, t, dis_col, w)


def _tc6_body(c01_ref, t_ref, dis_ref, h_ref, w22_ref, w30_ref, b3_ref,
              h2_out, c0b_out):
    tx2 = (2.0 * dis_ref[...]) * (t_ref[0] + t_ref[1]) - h_ref[...]
    h2 = jnp.maximum(
        c01_ref[...] + jnp.dot(tx2, w22_ref[...],
                               preferred_element_type=jnp.float32), 0.0)
    h2_out[...] = h2
    c0b_out[...] = jnp.dot(h2, w30_ref[...],
                           preferred_element_type=jnp.float32) + b3_ref[...]


def _tc6(c01, t2, dis_col, h, w22, w30, b3):
    return pl.pallas_call(
        _tc6_body,
        out_shape=[
            jax.ShapeDtypeStruct((NP, F), jnp.float32),
            jax.ShapeDtypeStruct((NP, 128), jnp.float32),
        ],
    )(c01, t2, dis_col, h, w22, w30, b3)


def _tc8_body(c01_ref, t_ref, dis_ref, h2_ref, w32_ref, batch_ref, wl_ref,
              bl_ref, out_ref):
    tx2 = (2.0 * dis_ref[...]) * (t_ref[0] + t_ref[1]) - h2_ref[...]
    h3 = jnp.maximum(
        c01_ref[...] + jnp.dot(tx2, w32_ref[...],
                               preferred_element_type=jnp.float32), 0.0)
    gids = lax.broadcasted_iota(jnp.int32, (1, G), 1)
    oh = (batch_ref[...] == gids).astype(jnp.float32)
    seg = lax.dot_general(oh, h3, (((0,), (0,)), ((), ())),
                          preferred_element_type=jnp.float32)
    cnt = jnp.sum(oh, axis=0)
    pooled = seg / jnp.maximum(cnt, 1.0)[:, None]
    out_ref[...] = jnp.dot(pooled, wl_ref[...],
                           preferred_element_type=jnp.float32) + bl_ref[...]


def _tc8(c01b, u2, dis_col, h2, w32, batchp, wl, bl):
    return pl.pallas_call(
        _tc8_body,
        out_shape=jax.ShapeDtypeStruct((G, wl.shape[-1]), jnp.float32),
    )(c01b, u2, dis_col, h2, w32, batchp, wl, bl)


def kernel(x, edge_index, edge_attr, batch, W1, b1, W2, b2, W3, b3, Wl, bl):
    src = edge_index[0].astype(jnp.int32)
    dst = edge_index[1].astype(jnp.int32)
    w = edge_attr.astype(jnp.float32)
    xp = jnp.pad(x, ((0, NP - N), (0, 0)))
    batchp = jnp.pad(batch.astype(jnp.int32), (0, NP - N),
                     constant_values=G).reshape(NP, 1)
    zeros1 = jnp.zeros((NP,), jnp.float32)
    zeros2 = jnp.zeros((NP, F), jnp.float32)
    w1cat = jnp.concatenate([W1[0], W1[1], W1[2]], axis=1)

    degp, wbar = _deg_kernel(src, dst, w, zeros1)
    dis80, a1, y1, y2 = _tc1(xp, w1cat, b1.reshape(1, F),
                             degp.reshape(NC, NP // 128, 128))
    dis = dis80.reshape(NP)
    dis_col = dis80.reshape(NP, 1)

    u, norm = _spmm_first(y2, src, dst, wbar, dis, zeros2)
    v = _tc3(y1, u, dis_col)
    sp = _spmm_next(v, src, dst, norm, zeros2)
    h, c0 = _tc4(a1, sp, dis_col, W2[0], b2.reshape(1, F))

    t1 = _spmm_next(h, src, dst, norm, zeros2)
    tx1, c01 = _tc5(c0, t1, dis_col, W2[1])
    t2 = _spmm_next(tx1, src, dst, norm, zeros2)
    h2, c0b = _tc6(c01, t2, dis_col, h, W2[2], W3[0], b3.reshape(1, 128))

    u1 = _spmm_next(h2, src, dst, norm, zeros2)
    u1s, c01b = _tc5(c0b, u1, dis_col, W3[1])
    u2 = _spmm_next(u1s, src, dst, norm, zeros2)
    return _tc8(c01b, u2, dis_col, h2, W3[2], batchp, Wl,
                bl.reshape(1, Wl.shape[-1]))


# speedup vs baseline: 14.3701x; 1.3378x over previous
import dataclasses
import functools

import jax
import jax.numpy as jnp
from jax import lax
from jax.experimental import pallas as pl
from jax.experimental.pallas import tpu as pltpu
from jax.experimental.pallas import tpu_sc as plsc

N = 10000
NP = 10240
E = 320000
G = 8
NC = 2
NS = 16
NW = NC * NS
EW = E // NW
B = 400
NB = EW // B
RPS = NP // NS
F = 64

_mesh = plsc.VectorSubcoreMesh(core_axis_name="c", subcore_axis_name="s")

_sc_params = pltpu.CompilerParams()
if "needs_layout_passes" in pltpu.CompilerParams.__dataclass_fields__:
    _sc_params = dataclasses.replace(_sc_params, needs_layout_passes=False)
if "use_tc_tiling_on_sc" in pltpu.CompilerParams.__dataclass_fields__:
    _sc_params = dataclasses.replace(_sc_params, use_tc_tiling_on_sc=False)


def _deg_body(src_hbm, dst_hbm, w_hbm, zeros_hbm, deg_out, wbar_out,
              sbig, dbig, wbig, si, acc):
    c = lax.axis_index("c")
    s = lax.axis_index("s")
    wid = c * NS + s
    rowbase = s * RPS
    ebase = wid * EW
    pltpu.sync_copy(zeros_hbm.at[pl.ds(rowbase, RPS)],
                    acc.at[pl.ds(rowbase, RPS)])
    pltpu.sync_copy(src_hbm.at[pl.ds(ebase, EW)], sbig)
    pltpu.sync_copy(dst_hbm.at[pl.ds(ebase, EW)], dbig)
    pltpu.sync_copy(w_hbm.at[pl.ds(ebase, EW)], wbig)
    plsc.subcore_barrier()

    @pl.loop(0, EW // 16, unroll=8)
    def _(g):
        sl = pl.ds(g * 16, 16)
        wbig[sl] = jnp.where(sbig[sl] != dbig[sl], wbig[sl], 0.0)

    pltpu.sync_copy(wbig, wbar_out.at[pl.ds(ebase, EW)])

    @pl.loop(0, NB)
    def _(blk):
        e0 = blk * B
        pltpu.sync_copy(src_hbm.at[pl.ds(ebase + e0, B)], si)
        pltpu.sync_copy(wbig.at[pl.ds(e0, B)], acc.at[si], add=True)

    plsc.subcore_barrier()
    pltpu.sync_copy(acc.at[pl.ds(rowbase, RPS)],
                    deg_out.at[c, pl.ds(rowbase, RPS)])


def _deg_kernel(src, dst, w, zeros1):
    return pl.kernel(
        _deg_body,
        out_type=[jax.ShapeDtypeStruct((NC, NP), jnp.float32),
                  jax.ShapeDtypeStruct((E,), jnp.float32)],
        mesh=_mesh,
        scratch_types=[
            pltpu.VMEM((EW,), jnp.int32),
            pltpu.VMEM((EW,), jnp.int32),
            pltpu.VMEM((EW,), jnp.float32),
            pltpu.VMEM((B,), jnp.int32),
            pltpu.VMEM_SHARED((NP,), jnp.float32),
        ],
        compiler_params=_sc_params,
    )(src, dst, w, zeros1)


def _spmm_body(compute_norm, z_hbm, src_hbm, dst_hbm, nrm_hbm, *rest):
    if compute_norm:
        (dis_hbm, zeros_hbm, out, norm_out,
         sbig, nbig, di0, di1, rows0, rows1, disv, acc,
         gsem0, gsem1, dsem0, dsem1, ssem0, ssem1) = rest
    else:
        (zeros_hbm, out,
         sbig, nbig, di0, di1, rows0, rows1, acc,
         gsem0, gsem1, dsem0, dsem1, ssem0, ssem1) = rest
    di = (di0, di1)
    rows = (rows0, rows1)
    gsem = (gsem0, gsem1)
    dsem = (dsem0, dsem1)
    ssem = (ssem0, ssem1)
    c = lax.axis_index("c")
    s = lax.axis_index("s")
    wid = c * NS + s
    rowbase = s * RPS
    ebase = wid * EW
    pltpu.sync_copy(zeros_hbm.at[pl.ds(rowbase, RPS)],
                    acc.at[pl.ds(rowbase, RPS)])
    pltpu.sync_copy(src_hbm.at[pl.ds(ebase, EW)], sbig)
    pltpu.sync_copy(nrm_hbm.at[pl.ds(ebase, EW)], nbig)
    if compute_norm:
        pltpu.sync_copy(dis_hbm, disv)
    plsc.subcore_barrier()

    if compute_norm:
        @pl.loop(0, EW // 16, unroll=8)
        def _(g):
            sl = pl.ds(g * 16, 16)
            nbig[sl] = -nbig[sl] * plsc.load_gather(disv, [sbig[sl]])

        pltpu.sync_copy(nbig, norm_out.at[pl.ds(ebase, EW)])

    def issue_in(k, p):
        pltpu.async_copy(dst_hbm.at[pl.ds(ebase + k * B, B)], di[p],
                         dsem[p])
        pltpu.async_copy(z_hbm.at[sbig.at[pl.ds(k * B, B)]], rows[p],
                         gsem[p])

    def wait_in(p):
        pltpu.make_async_copy(dst_hbm.at[pl.ds(ebase, B)], di[p],
                              dsem[p]).wait()
        pltpu.make_async_copy(z_hbm.at[sbig.at[pl.ds(0, B)]], rows[p],
                              gsem[p]).wait()

    def issue_scat(p):
        pltpu.async_copy(rows[p], acc.at[di[p]], ssem[p], add=True)

    def wait_scat(p):
        pltpu.make_async_copy(rows[p], acc.at[di[p]], ssem[p]).wait()

    def scale(k, p):
        e0 = k * B

        @pl.loop(0, B, unroll=8)
        def _(i):
            n16 = plsc.load_gather(
                nbig, [jnp.zeros((16,), jnp.int32) + (e0 + i)])
            row = rows[p].at[i]
            for ch in range(F // 16):
                cs = pl.ds(ch * 16, 16)
                row[cs] = row[cs] * n16

    issue_in(0, 0)
    wait_in(0)
    issue_in(1, 1)
    scale(0, 0)
    issue_scat(0)

    @pl.loop(0, (NB - 3) // 2)
    def _(t):
        kA = 2 * t + 1
        wait_in(1)
        wait_scat(0)
        issue_in(kA + 1, 0)
        scale(kA, 1)
        issue_scat(1)
        wait_in(0)
        wait_scat(1)
        issue_in(kA + 2, 1)
        scale(kA + 1, 0)
        issue_scat(0)

    wait_in(1)
    wait_scat(0)
    issue_in(NB - 1, 0)
    scale(NB - 2, 1)
    issue_scat(1)
    wait_in(0)
    wait_scat(1)
    scale(NB - 1, 0)
    issue_scat(0)
    wait_scat(0)

    plsc.subcore_barrier()
    pltpu.sync_copy(acc.at[pl.ds(rowbase, RPS)],
                    out.at[c, pl.ds(rowbase, RPS)])


def _make_spmm(compute_norm):
    if compute_norm:
        out_type = [jax.ShapeDtypeStruct((NC, NP, F), jnp.float32),
                    jax.ShapeDtypeStruct((E,), jnp.float32)]
    else:
        out_type = jax.ShapeDtypeStruct((NC, NP, F), jnp.float32)
    scratch = [
        pltpu.VMEM((EW,), jnp.int32),
        pltpu.VMEM((EW,), jnp.float32),
        pltpu.VMEM((B,), jnp.int32),
        pltpu.VMEM((B,), jnp.int32),
        pltpu.VMEM((B, F), jnp.float32),
        pltpu.VMEM((B, F), jnp.float32),
    ]
    if compute_norm:
        scratch.append(pltpu.VMEM((NP,), jnp.float32))
    scratch.append(pltpu.VMEM_SHARED((NP, F), jnp.float32))
    scratch.extend([pltpu.SemaphoreType.DMA] * 6)
    return pl.kernel(
        functools.partial(_spmm_body, compute_norm),
        out_type=out_type,
        mesh=_mesh,
        scratch_types=scratch,
        compiler_params=_sc_params,
    )


_spmm_first = _make_spmm(True)
_spmm_next = _make_spmm(False)


def _tc1_body(x_ref, w_ref, b_ref, deg_ref, dis_out, a1_out, y1_out, y2_out):
    deg = deg_ref[0] + deg_ref[1]
    dis_out[...] = jnp.where(deg > 0, lax.rsqrt(jnp.where(deg > 0, deg, 1.0)),
                             0.0)
    y = jnp.dot(x_ref[...], w_ref[...], preferred_element_type=jnp.float32)
    y2 = y[:, 2 * F:]
    a1_out[...] = y[:, :F] - y2 + b_ref[...]
    y1_out[...] = y[:, F:2 * F]
    y2_out[...] = y2


def _tc1(xp, w1cat, b1, degp):
    return pl.pallas_call(
        _tc1_body,
        out_shape=[
            jax.ShapeDtypeStruct((NP // 128, 128), jnp.float32),
            jax.ShapeDtypeStruct((NP, F), jnp.float32),
            jax.ShapeDtypeStruct((NP, F), jnp.float32),
            jax.ShapeDtypeStruct((NP, F), jnp.float32),
        ],
    )(xp, w1cat, b1, degp)


def _tc3_body(y1_ref, u_ref, dis_ref, v_out):
    v_out[...] = y1_ref[...] + (2.0 * dis_ref[...]) * (u_ref[0] + u_ref[1])


def _tc3(y1, u, dis_col):
    return pl.pallas_call(
        _tc3_body,
        out_shape=jax.ShapeDtypeStruct((NP, F), jnp.float32),
    )(y1, u, dis_col)


def _tc4_body(a1_ref, sp_ref, dis_ref, w_ref, b_ref, h_out, c0_out):
    h = jnp.maximum(a1_ref[...] + dis_ref[...] * (sp_ref[0] + sp_ref[1]), 0.0)
    h_out[...] = h
    c0_out[...] = jnp.dot(h, w_ref[...],
                          preferred_element_type=jnp.float32) + b_ref[...]


def _tc4(a1, sp, dis_col, w20, b2):
    return pl.pallas_call(
        _tc4_body,
        out_shape=[
            jax.ShapeDtypeStruct((NP, F), jnp.float32),
            jax.ShapeDtypeStruct((NP, F), jnp.float32),
        ],
    )(a1, sp, dis_col, w20, b2)


def _tc5_body(c0_ref, t_ref, dis_ref, w_ref, tx_out, c01_out):
    tx = dis_ref[...] * (t_ref[0] + t_ref[1])
    tx_out[...] = tx
    c01_out[...] = c0_ref[...] + jnp.dot(tx, w_ref[...],
                                         preferred_element_type=jnp.float32)


def _tc5(c0, t, dis_col, w):
    fo = w.shape[-1]
    return pl.pallas_call(
        _tc5_body,
        out_shape=[
            jax.ShapeDtypeStruct((NP, F), jnp.float32),
            jax.ShapeDtypeStruct((NP, fo), jnp.float32),
        ],
    )(c0, t, dis_col, w)


def _tc6_body(c01_ref, t_ref, dis_ref, h_ref, w22_ref, w30_ref, b3_ref,
              h2_out, c0b_out):
    tx2 = (2.0 * dis_ref[...]) * (t_ref[0] + t_ref[1]) - h_ref[...]
    h2 = jnp.maximum(
        c01_ref[...] + jnp.dot(tx2, w22_ref[...],
                               preferred_element_type=jnp.float32), 0.0)
    h2_out[...] = h2
    c0b_out[...] = jnp.dot(h2, w30_ref[...],
                           preferred_element_type=jnp.float32) + b3_ref[...]


def _tc6(c01, t2, dis_col, h, w22, w30, b3):
    return pl.pallas_call(
        _tc6_body,
        out_shape=[
            jax.ShapeDtypeStruct((NP, F), jnp.float32),
            jax.ShapeDtypeStruct((NP, 128), jnp.float32),
        ],
    )(c01, t2, dis_col, h, w22, w30, b3)


def _tc8_body(c01_ref, t_ref, dis_ref, h2_ref, w32_ref, batch_ref, wl_ref,
              bl_ref, out_ref):
    tx2 = (2.0 * dis_ref[...]) * (t_ref[0] + t_ref[1]) - h2_ref[...]
    h3 = jnp.maximum(
        c01_ref[...] + jnp.dot(tx2, w32_ref[...],
                               preferred_element_type=jnp.float32), 0.0)
    gids = lax.broadcasted_iota(jnp.int32, (1, G), 1)
    oh = (batch_ref[...] == gids).astype(jnp.float32)
    seg = lax.dot_general(oh, h3, (((0,), (0,)), ((), ())),
                          preferred_element_type=jnp.float32)
    cnt = jnp.sum(oh, axis=0)
    pooled = seg / jnp.maximum(cnt, 1.0)[:, None]
    out_ref[...] = jnp.dot(pooled, wl_ref[...],
                           preferred_element_type=jnp.float32) + bl_ref[...]


def _tc8(c01b, u2, dis_col, h2, w32, batchp, wl, bl):
    return pl.pallas_call(
        _tc8_body,
        out_shape=jax.ShapeDtypeStruct((G, wl.shape[-1]), jnp.float32),
    )(c01b, u2, dis_col, h2, w32, batchp, wl, bl)


def kernel(x, edge_index, edge_attr, batch, W1, b1, W2, b2, W3, b3, Wl, bl):
    src = edge_index[0].astype(jnp.int32)
    dst = edge_index[1].astype(jnp.int32)
    w = edge_attr.astype(jnp.float32)
    xp = jnp.pad(x, ((0, NP - N), (0, 0)))
    batchp = jnp.pad(batch.astype(jnp.int32), (0, NP - N),
                     constant_values=G).reshape(NP, 1)
    zeros1 = jnp.zeros((NP,), jnp.float32)
    zeros2 = jnp.zeros((NP, F), jnp.float32)
    w1cat = jnp.concatenate([W1[0], W1[1], W1[2]], axis=1)

    degp, wbar = _deg_kernel(src, dst, w, zeros1)
    dis80, a1, y1, y2 = _tc1(xp, w1cat, b1.reshape(1, F),
                             degp.reshape(NC, NP // 128, 128))
    dis = dis80.reshape(NP)
    dis_col = dis80.reshape(NP, 1)

    u, norm = _spmm_first(y2, src, dst, wbar, dis, zeros2)
    v = _tc3(y1, u, dis_col)
    sp = _spmm_next(v, src, dst, norm, zeros2)
    h, c0 = _tc4(a1, sp, dis_col, W2[0], b2.reshape(1, F))

    t1 = _spmm_next(h, src, dst, norm, zeros2)
    tx1, c01 = _tc5(c0, t1, dis_col, W2[1])
    t2 = _spmm_next(tx1, src, dst, norm, zeros2)
    h2, c0b = _tc6(c01, t2, dis_col, h, W2[2], W3[0], b3.reshape(1, 128))

    u1 = _spmm_next(h2, src, dst, norm, zeros2)
    u1s, c01b = _tc5(c0b, u1, dis_col, W3[1])
    u2 = _spmm_next(u1s, src, dst, norm, zeros2)
    return _tc8(c01b, u2, dis_col, h2, W3[2], batchp, Wl,
                bl.reshape(1, Wl.shape[-1]))


# speedup vs baseline: 14.4086x; 1.0027x over previous
import dataclasses
import functools

import jax
import jax.numpy as jnp
from jax import lax
from jax.experimental import pallas as pl
from jax.experimental.pallas import tpu as pltpu
from jax.experimental.pallas import tpu_sc as plsc

N = 10000
NP = 10240
E = 320000
G = 8
NC = 2
NS = 16
NW = NC * NS
EW = E // NW
B = 80
NB = EW // B
RPS = NP // NS
F = 64

_mesh = plsc.VectorSubcoreMesh(core_axis_name="c", subcore_axis_name="s")

_sc_params = pltpu.CompilerParams()
if "needs_layout_passes" in pltpu.CompilerParams.__dataclass_fields__:
    _sc_params = dataclasses.replace(_sc_params, needs_layout_passes=False)
if "use_tc_tiling_on_sc" in pltpu.CompilerParams.__dataclass_fields__:
    _sc_params = dataclasses.replace(_sc_params, use_tc_tiling_on_sc=False)


def _deg_body(src_hbm, dst_hbm, w_hbm, zeros_hbm, deg_out, wbar_out,
              sbig, dbig, wbig, si, acc):
    c = lax.axis_index("c")
    s = lax.axis_index("s")
    wid = c * NS + s
    rowbase = s * RPS
    ebase = wid * EW
    pltpu.sync_copy(zeros_hbm.at[pl.ds(rowbase, RPS)],
                    acc.at[pl.ds(rowbase, RPS)])
    pltpu.sync_copy(src_hbm.at[pl.ds(ebase, EW)], sbig)
    pltpu.sync_copy(dst_hbm.at[pl.ds(ebase, EW)], dbig)
    pltpu.sync_copy(w_hbm.at[pl.ds(ebase, EW)], wbig)
    plsc.subcore_barrier()

    @pl.loop(0, EW // 16, unroll=8)
    def _(g):
        sl = pl.ds(g * 16, 16)
        wbig[sl] = jnp.where(sbig[sl] != dbig[sl], wbig[sl], 0.0)

    pltpu.sync_copy(wbig, wbar_out.at[pl.ds(ebase, EW)])

    @pl.loop(0, NB)
    def _(blk):
        e0 = blk * B
        pltpu.sync_copy(src_hbm.at[pl.ds(ebase + e0, B)], si)
        pltpu.sync_copy(wbig.at[pl.ds(e0, B)], acc.at[si], add=True)

    plsc.subcore_barrier()
    pltpu.sync_copy(acc.at[pl.ds(rowbase, RPS)],
                    deg_out.at[c, pl.ds(rowbase, RPS)])


def _deg_kernel(src, dst, w, zeros1):
    return pl.kernel(
        _deg_body,
        out_type=[jax.ShapeDtypeStruct((NC, NP), jnp.float32),
                  jax.ShapeDtypeStruct((E,), jnp.float32)],
        mesh=_mesh,
        scratch_types=[
            pltpu.VMEM((EW,), jnp.int32),
            pltpu.VMEM((EW,), jnp.int32),
            pltpu.VMEM((EW,), jnp.float32),
            pltpu.VMEM((B,), jnp.int32),
            pltpu.VMEM_SHARED((NP,), jnp.float32),
        ],
        compiler_params=_sc_params,
    )(src, dst, w, zeros1)


def _spmm_body(compute_norm, z_hbm, src_hbm, dst_hbm, nrm_hbm, *rest):
    if compute_norm:
        (dis_hbm, zeros_hbm, out, norm_out,
         sbig, nbig, di0, di1, di2, rows0, rows1, rows2, disv, acc,
         gsem0, gsem1, gsem2, dsem0, dsem1, dsem2,
         ssem0, ssem1, ssem2) = rest
    else:
        (zeros_hbm, out,
         sbig, nbig, di0, di1, di2, rows0, rows1, rows2, acc,
         gsem0, gsem1, gsem2, dsem0, dsem1, dsem2,
         ssem0, ssem1, ssem2) = rest
    di = (di0, di1, di2)
    rows = (rows0, rows1, rows2)
    gsem = (gsem0, gsem1, gsem2)
    dsem = (dsem0, dsem1, dsem2)
    ssem = (ssem0, ssem1, ssem2)
    c = lax.axis_index("c")
    s = lax.axis_index("s")
    wid = c * NS + s
    rowbase = s * RPS
    ebase = wid * EW
    pltpu.sync_copy(zeros_hbm.at[pl.ds(rowbase, RPS)],
                    acc.at[pl.ds(rowbase, RPS)])
    pltpu.sync_copy(src_hbm.at[pl.ds(ebase, EW)], sbig)
    pltpu.sync_copy(nrm_hbm.at[pl.ds(ebase, EW)], nbig)
    if compute_norm:
        pltpu.sync_copy(dis_hbm, disv)
    plsc.subcore_barrier()

    if compute_norm:
        @pl.loop(0, EW // 16, unroll=8)
        def _(g):
            sl = pl.ds(g * 16, 16)
            nbig[sl] = -nbig[sl] * plsc.load_gather(disv, [sbig[sl]])

        pltpu.sync_copy(nbig, norm_out.at[pl.ds(ebase, EW)])

    def issue_in(k, p):
        pltpu.async_copy(dst_hbm.at[pl.ds(ebase + k * B, B)], di[p],
                         dsem[p])
        pltpu.async_copy(z_hbm.at[sbig.at[pl.ds(k * B, B)]], rows[p],
                         gsem[p])

    def wait_in(p):
        pltpu.make_async_copy(dst_hbm.at[pl.ds(ebase, B)], di[p],
                              dsem[p]).wait()
        pltpu.make_async_copy(z_hbm.at[sbig.at[pl.ds(0, B)]], rows[p],
                              gsem[p]).wait()

    def issue_scat(p):
        pltpu.async_copy(rows[p], acc.at[di[p]], ssem[p], add=True)

    def wait_scat(p):
        pltpu.make_async_copy(rows[p], acc.at[di[p]], ssem[p]).wait()

    def scale(k, p):
        e0 = k * B

        @pl.loop(0, B, unroll=8)
        def _(i):
            n16 = plsc.load_gather(
                nbig, [jnp.zeros((16,), jnp.int32) + (e0 + i)])
            row = rows[p].at[i]
            for ch in range(F // 16):
                cs = pl.ds(ch * 16, 16)
                row[cs] = row[cs] * n16

    def body(k, p):
        wait_in(p)
        scale(k, p)
        issue_scat(p)
        q = (p + 2) % 3
        wait_scat(q)
        issue_in(k + 2, q)

    issue_in(0, 0)
    issue_in(1, 1)
    wait_in(0)
    scale(0, 0)
    issue_scat(0)
    issue_in(2, 2)

    nfull = NB - 3
    nloop = nfull // 3

    @pl.loop(0, nloop)
    def _(t):
        k = 3 * t + 1
        body(k, 1)
        body(k + 1, 2)
        body(k + 2, 0)

    for j in range(nfull % 3):
        kj = 3 * nloop + 1 + j
        body(kj, kj % 3)
    for kj in (NB - 2, NB - 1):
        wait_in(kj % 3)
        scale(kj, kj % 3)
        issue_scat(kj % 3)
    wait_scat((NB - 3) % 3)
    wait_scat((NB - 2) % 3)
    wait_scat((NB - 1) % 3)

    plsc.subcore_barrier()
    pltpu.sync_copy(acc.at[pl.ds(rowbase, RPS)],
                    out.at[c, pl.ds(rowbase, RPS)])


def _make_spmm(compute_norm):
    if compute_norm:
        out_type = [jax.ShapeDtypeStruct((NC, NP, F), jnp.float32),
                    jax.ShapeDtypeStruct((E,), jnp.float32)]
    else:
        out_type = jax.ShapeDtypeStruct((NC, NP, F), jnp.float32)
    scratch = [
        pltpu.VMEM((EW,), jnp.int32),
        pltpu.VMEM((EW,), jnp.float32),
        pltpu.VMEM((B,), jnp.int32),
        pltpu.VMEM((B,), jnp.int32),
        pltpu.VMEM((B,), jnp.int32),
        pltpu.VMEM((B, F), jnp.float32),
        pltpu.VMEM((B, F), jnp.float32),
        pltpu.VMEM((B, F), jnp.float32),
    ]
    if compute_norm:
        scratch.append(pltpu.VMEM((NP,), jnp.float32))
    scratch.append(pltpu.VMEM_SHARED((NP, F), jnp.float32))
    scratch.extend([pltpu.SemaphoreType.DMA] * 9)
    return pl.kernel(
        functools.partial(_spmm_body, compute_norm),
        out_type=out_type,
        mesh=_mesh,
        scratch_types=scratch,
        compiler_params=_sc_params,
    )


_spmm_first = _make_spmm(True)
_spmm_next = _make_spmm(False)


def _tc1_body(x_ref, w_ref, b_ref, deg_ref, dis_out, a1_out, y1_out, y2_out):
    deg = deg_ref[0] + deg_ref[1]
    dis_out[...] = jnp.where(deg > 0, lax.rsqrt(jnp.where(deg > 0, deg, 1.0)),
                             0.0)
    y = jnp.dot(x_ref[...], w_ref[...], preferred_element_type=jnp.float32)
    y2 = y[:, 2 * F:]
    a1_out[...] = y[:, :F] - y2 + b_ref[...]
    y1_out[...] = y[:, F:2 * F]
    y2_out[...] = y2


def _tc1(xp, w1cat, b1, degp):
    return pl.pallas_call(
        _tc1_body,
        out_shape=[
            jax.ShapeDtypeStruct((NP // 128, 128), jnp.float32),
            jax.ShapeDtypeStruct((NP, F), jnp.float32),
            jax.ShapeDtypeStruct((NP, F), jnp.float32),
            jax.ShapeDtypeStruct((NP, F), jnp.float32),
        ],
    )(xp, w1cat, b1, degp)


def _tc3_body(y1_ref, u_ref, dis_ref, v_out):
    v_out[...] = y1_ref[...] + (2.0 * dis_ref[...]) * (u_ref[0] + u_ref[1])


def _tc3(y1, u, dis_col):
    return pl.pallas_call(
        _tc3_body,
        out_shape=jax.ShapeDtypeStruct((NP, F), jnp.float32),
    )(y1, u, dis_col)


def _tc4_body(a1_ref, sp_ref, dis_ref, w_ref, b_ref, h_out, c0_out):
    h = jnp.maximum(a1_ref[...] + dis_ref[...] * (sp_ref[0] + sp_ref[1]), 0.0)
    h_out[...] = h
    c0_out[...] = jnp.dot(h, w_ref[...],
                          preferred_element_type=jnp.float32) + b_ref[...]


def _tc4(a1, sp, dis_col, w20, b2):
    return pl.pallas_call(
        _tc4_body,
        out_shape=[
            jax.ShapeDtypeStruct((NP, F), jnp.float32),
            jax.ShapeDtypeStruct((NP, F), jnp.float32),
        ],
    )(a1, sp, dis_col, w20, b2)


def _tc5_body(c0_ref, t_ref, dis_ref, w_ref, tx_out, c01_out):
    tx = dis_ref[...] * (t_ref[0] + t_ref[1])
    tx_out[...] = tx
    c01_out[...] = c0_ref[...] + jnp.dot(tx, w_ref[...],
                                         preferred_element_type=jnp.float32)


def _tc5(c0, t, dis_col, w):
    fo = w.shape[-1]
    return pl.pallas_call(
        _tc5_body,
        out_shape=[
            jax.ShapeDtypeStruct((NP, F), jnp.float32),
            jax.ShapeDtypeStruct((NP, fo), jnp.float32),
        ],
    )(c0, t, dis_col, w)


def _tc6_body(c01_ref, t_ref, dis_ref, h_ref, w22_ref, w30_ref, b3_ref,
              h2_out, c0b_out):
    tx2 = (2.0 * dis_ref[...]) * (t_ref[0] + t_ref[1]) - h_ref[...]
    h2 = jnp.maximum(
        c01_ref[...] + jnp.dot(tx2, w22_ref[...],
                               preferred_element_type=jnp.float32), 0.0)
    h2_out[...] = h2
    c0b_out[...] = jnp.dot(h2, w30_ref[...],
                           preferred_element_type=jnp.float32) + b3_ref[...]


def _tc6(c01, t2, dis_col, h, w22, w30, b3):
    return pl.pallas_call(
        _tc6_body,
        out_shape=[
            jax.ShapeDtypeStruct((NP, F), jnp.float32),
            jax.ShapeDtypeStruct((NP, 128), jnp.float32),
        ],
    )(c01, t2, dis_col, h, w22, w30, b3)


def _tc8_body(c01_ref, t_ref, dis_ref, h2_ref, w32_ref, batch_ref, wl_ref,
              bl_ref, out_ref):
    tx2 = (2.0 * dis_ref[...]) * (t_ref[0] + t_ref[1]) - h2_ref[...]
    h3 = jnp.maximum(
        c01_ref[...] + jnp.dot(tx2, w32_ref[...],
                               preferred_element_type=jnp.float32), 0.0)
    gids = lax.broadcasted_iota(jnp.int32, (1, G), 1)
    oh = (batch_ref[...] == gids).astype(jnp.float32)
    seg = lax.dot_general(oh, h3, (((0,), (0,)), ((), ())),
                          preferred_element_type=jnp.float32)
    cnt = jnp.sum(oh, axis=0)
    pooled = seg / jnp.maximum(cnt, 1.0)[:, None]
    out_ref[...] = jnp.dot(pooled, wl_ref[...],
                           preferred_element_type=jnp.float32) + bl_ref[...]


def _tc8(c01b, u2, dis_col, h2, w32, batchp, wl, bl):
    return pl.pallas_call(
        _tc8_body,
        out_shape=jax.ShapeDtypeStruct((G, wl.shape[-1]), jnp.float32),
    )(c01b, u2, dis_col, h2, w32, batchp, wl, bl)


def kernel(x, edge_index, edge_attr, batch, W1, b1, W2, b2, W3, b3, Wl, bl):
    src = edge_index[0].astype(jnp.int32)
    dst = edge_index[1].astype(jnp.int32)
    w = edge_attr.astype(jnp.float32)
    xp = jnp.pad(x, ((0, NP - N), (0, 0)))
    batchp = jnp.pad(batch.astype(jnp.int32), (0, NP - N),
                     constant_values=G).reshape(NP, 1)
    zeros1 = jnp.zeros((NP,), jnp.float32)
    zeros2 = jnp.zeros((NP, F), jnp.float32)
    w1cat = jnp.concatenate([W1[0], W1[1], W1[2]], axis=1)

    degp, wbar = _deg_kernel(src, dst, w, zeros1)
    dis80, a1, y1, y2 = _tc1(xp, w1cat, b1.reshape(1, F),
                             degp.reshape(NC, NP // 128, 128))
    dis = dis80.reshape(NP)
    dis_col = dis80.reshape(NP, 1)

    u, norm = _spmm_first(y2, src, dst, wbar, dis, zeros2)
    v = _tc3(y1, u, dis_col)
    sp = _spmm_next(v, src, dst, norm, zeros2)
    h, c0 = _tc4(a1, sp, dis_col, W2[0], b2.reshape(1, F))

    t1 = _spmm_next(h, src, dst, norm, zeros2)
    tx1, c01 = _tc5(c0, t1, dis_col, W2[1])
    t2 = _spmm_next(tx1, src, dst, norm, zeros2)
    h2, c0b = _tc6(c01, t2, dis_col, h, W2[2], W3[0], b3.reshape(1, 128))

    u1 = _spmm_next(h2, src, dst, norm, zeros2)
    u1s, c01b = _tc5(c0b, u1, dis_col, W3[1])
    u2 = _spmm_next(u1s, src, dst, norm, zeros2)
    return _tc8(c01b, u2, dis_col, h2, W3[2], batchp, Wl,
                bl.reshape(1, Wl.shape[-1]))


# speedup vs baseline: 15.2010x; 1.0550x over previous
import dataclasses
import functools

import jax
import jax.numpy as jnp
from jax import lax
from jax.experimental import pallas as pl
from jax.experimental.pallas import tpu as pltpu
from jax.experimental.pallas import tpu_sc as plsc

N = 10000
NP = 10240
E = 320000
G = 8
NC = 2
NS = 16
NW = NC * NS
EW = E // NW
B = 80
NB = EW // B
RPS = NP // NS
F = 64

_mesh = plsc.VectorSubcoreMesh(core_axis_name="c", subcore_axis_name="s")

_sc_params = pltpu.CompilerParams()
if "needs_layout_passes" in pltpu.CompilerParams.__dataclass_fields__:
    _sc_params = dataclasses.replace(_sc_params, needs_layout_passes=False)
if "use_tc_tiling_on_sc" in pltpu.CompilerParams.__dataclass_fields__:
    _sc_params = dataclasses.replace(_sc_params, use_tc_tiling_on_sc=False)


EPS = E // NS
B2 = 800
NB2 = EPS // B2


def _prep_body(src_hbm, dst_hbm, w_hbm, zeros_hbm, dis_out, norm_out,
               sbig, dbig, wbig, si, nv, dv, disv, acc, dis_sh):
    c = lax.axis_index("c")
    s = lax.axis_index("s")
    wid = c * NS + s
    rowbase = s * RPS
    ebase = s * EPS
    pltpu.sync_copy(zeros_hbm.at[pl.ds(rowbase, RPS)],
                    acc.at[pl.ds(rowbase, RPS)])
    pltpu.sync_copy(src_hbm.at[pl.ds(ebase, EPS)], sbig)
    pltpu.sync_copy(dst_hbm.at[pl.ds(ebase, EPS)], dbig)
    pltpu.sync_copy(w_hbm.at[pl.ds(ebase, EPS)], wbig)
    plsc.subcore_barrier()

    @pl.loop(0, EPS // 16, unroll=8)
    def _(g):
        sl = pl.ds(g * 16, 16)
        wbig[sl] = jnp.where(sbig[sl] != dbig[sl], wbig[sl], 0.0)

    @pl.loop(0, NB2)
    def _(blk):
        e0 = blk * B2
        pltpu.sync_copy(src_hbm.at[pl.ds(ebase + e0, B2)], si)
        pltpu.sync_copy(wbig.at[pl.ds(e0, B2)], acc.at[si], add=True)

    plsc.subcore_barrier()
    pltpu.sync_copy(acc.at[pl.ds(rowbase, RPS)], dv)

    @pl.loop(0, RPS // 16, unroll=4)
    def _(g):
        sl = pl.ds(g * 16, 16)
        x = dv[sl]
        i = plsc.bitcast(x, jnp.int32)
        y = plsc.bitcast(jnp.int32(0x5F3759DF) - lax.shift_right_logical(
            i, 1), jnp.float32)
        for _ in range(4):
            y = y * (1.5 - 0.5 * x * y * y)
        dv[sl] = jnp.where(x > 0, y, 0.0)

    hw = RPS // NC
    pltpu.sync_copy(dv.at[pl.ds(c * hw, hw)],
                    dis_out.at[pl.ds(rowbase + c * hw, hw)])
    pltpu.sync_copy(dv, dis_sh.at[pl.ds(rowbase, RPS)])
    plsc.subcore_barrier()
    pltpu.sync_copy(dis_sh, disv)

    wbase = wid * EW
    pltpu.sync_copy(src_hbm.at[pl.ds(wbase, EW)], sbig.at[pl.ds(0, EW)])
    pltpu.sync_copy(dst_hbm.at[pl.ds(wbase, EW)], dbig.at[pl.ds(0, EW)])
    pltpu.sync_copy(w_hbm.at[pl.ds(wbase, EW)], wbig.at[pl.ds(0, EW)])

    @pl.loop(0, EW // 16, unroll=8)
    def _(g):
        sl = pl.ds(g * 16, 16)
        wm = jnp.where(sbig[sl] != dbig[sl], wbig[sl], 0.0)
        nv[sl] = -wm * plsc.load_gather(disv, [sbig[sl]])

    pltpu.sync_copy(nv, norm_out.at[pl.ds(wbase, EW)])


def _prep_kernel(src, dst, w, zeros1):
    return pl.kernel(
        _prep_body,
        out_type=[jax.ShapeDtypeStruct((NP,), jnp.float32),
                  jax.ShapeDtypeStruct((E,), jnp.float32)],
        mesh=_mesh,
        scratch_types=[
            pltpu.VMEM((EPS,), jnp.int32),
            pltpu.VMEM((EPS,), jnp.int32),
            pltpu.VMEM((EPS,), jnp.float32),
            pltpu.VMEM((B2,), jnp.int32),
            pltpu.VMEM((EW,), jnp.float32),
            pltpu.VMEM((RPS,), jnp.float32),
            pltpu.VMEM((NP,), jnp.float32),
            pltpu.VMEM_SHARED((NP,), jnp.float32),
            pltpu.VMEM_SHARED((NP,), jnp.float32),
        ],
        compiler_params=_sc_params,
    )(src, dst, w, zeros1)


def _spmm_body(z_hbm, src_hbm, dst_hbm, nrm_hbm, zeros_hbm, out,
               sbig, nbig, di0, di1, di2, rows0, rows1, rows2, acc,
               gsem0, gsem1, gsem2, dsem0, dsem1, dsem2,
               ssem0, ssem1, ssem2):
    di = (di0, di1, di2)
    rows = (rows0, rows1, rows2)
    gsem = (gsem0, gsem1, gsem2)
    dsem = (dsem0, dsem1, dsem2)
    ssem = (ssem0, ssem1, ssem2)
    c = lax.axis_index("c")
    s = lax.axis_index("s")
    wid = c * NS + s
    rowbase = s * RPS
    ebase = wid * EW
    pltpu.sync_copy(zeros_hbm.at[pl.ds(rowbase, RPS)],
                    acc.at[pl.ds(rowbase, RPS)])
    pltpu.sync_copy(src_hbm.at[pl.ds(ebase, EW)], sbig)
    pltpu.sync_copy(nrm_hbm.at[pl.ds(ebase, EW)], nbig)
    plsc.subcore_barrier()

    def issue_in(k, p):
        pltpu.async_copy(dst_hbm.at[pl.ds(ebase + k * B, B)], di[p],
                         dsem[p])
        pltpu.async_copy(z_hbm.at[sbig.at[pl.ds(k * B, B)]], rows[p],
                         gsem[p])

    def wait_in(p):
        pltpu.make_async_copy(dst_hbm.at[pl.ds(ebase, B)], di[p],
                              dsem[p]).wait()
        pltpu.make_async_copy(z_hbm.at[sbig.at[pl.ds(0, B)]], rows[p],
                              gsem[p]).wait()

    def issue_scat(p):
        pltpu.async_copy(rows[p], acc.at[di[p]], ssem[p], add=True)

    def wait_scat(p):
        pltpu.make_async_copy(rows[p], acc.at[di[p]], ssem[p]).wait()

    def scale(k, p):
        e0 = k * B

        @pl.loop(0, B, unroll=8)
        def _(i):
            n16 = plsc.load_gather(
                nbig, [jnp.zeros((16,), jnp.int32) + (e0 + i)])
            row = rows[p].at[i]
            for ch in range(F // 16):
                cs = pl.ds(ch * 16, 16)
                row[cs] = row[cs] * n16

    def body(k, p):
        wait_in(p)
        scale(k, p)
        issue_scat(p)
        q = (p + 2) % 3
        wait_scat(q)
        issue_in(k + 2, q)

    issue_in(0, 0)
    issue_in(1, 1)
    wait_in(0)
    scale(0, 0)
    issue_scat(0)
    issue_in(2, 2)

    nfull = NB - 3
    nloop = nfull // 3

    @pl.loop(0, nloop)
    def _(t):
        k = 3 * t + 1
        body(k, 1)
        body(k + 1, 2)
        body(k + 2, 0)

    for j in range(nfull % 3):
        kj = 3 * nloop + 1 + j
        body(kj, kj % 3)
    for kj in (NB - 2, NB - 1):
        wait_in(kj % 3)
        scale(kj, kj % 3)
        issue_scat(kj % 3)
    wait_scat((NB - 3) % 3)
    wait_scat((NB - 2) % 3)
    wait_scat((NB - 1) % 3)

    plsc.subcore_barrier()
    pltpu.sync_copy(acc.at[pl.ds(rowbase, RPS)],
                    out.at[c, pl.ds(rowbase, RPS)])


_spmm_next = pl.kernel(
    _spmm_body,
    out_type=jax.ShapeDtypeStruct((NC, NP, F), jnp.float32),
    mesh=_mesh,
    scratch_types=[
        pltpu.VMEM((EW,), jnp.int32),
        pltpu.VMEM((EW,), jnp.float32),
        pltpu.VMEM((B,), jnp.int32),
        pltpu.VMEM((B,), jnp.int32),
        pltpu.VMEM((B,), jnp.int32),
        pltpu.VMEM((B, F), jnp.float32),
        pltpu.VMEM((B, F), jnp.float32),
        pltpu.VMEM((B, F), jnp.float32),
        pltpu.VMEM_SHARED((NP, F), jnp.float32),
    ] + [pltpu.SemaphoreType.DMA] * 9,
    compiler_params=_sc_params,
)


def _tc1_body(x_ref, w_ref, b_ref, a1_out, y1_out, y2_out):
    y = jnp.dot(x_ref[...], w_ref[...], preferred_element_type=jnp.float32)
    y2 = y[:, 2 * F:]
    a1_out[...] = y[:, :F] - y2 + b_ref[...]
    y1_out[...] = y[:, F:2 * F]
    y2_out[...] = y2


def _tc1(xp, w1cat, b1):
    return pl.pallas_call(
        _tc1_body,
        out_shape=[
            jax.ShapeDtypeStruct((NP, F), jnp.float32),
            jax.ShapeDtypeStruct((NP, F), jnp.float32),
            jax.ShapeDtypeStruct((NP, F), jnp.float32),
        ],
    )(xp, w1cat, b1)


def _tc3_body(y1_ref, u_ref, dis_ref, v_out):
    v_out[...] = y1_ref[...] + (2.0 * dis_ref[...]) * (u_ref[0] + u_ref[1])


def _tc3(y1, u, dis_col):
    return pl.pallas_call(
        _tc3_body,
        out_shape=jax.ShapeDtypeStruct((NP, F), jnp.float32),
    )(y1, u, dis_col)


def _tc4_body(a1_ref, sp_ref, dis_ref, w_ref, b_ref, h_out, c0_out):
    h = jnp.maximum(a1_ref[...] + dis_ref[...] * (sp_ref[0] + sp_ref[1]), 0.0)
    h_out[...] = h
    c0_out[...] = jnp.dot(h, w_ref[...],
                          preferred_element_type=jnp.float32) + b_ref[...]


def _tc4(a1, sp, dis_col, w20, b2):
    return pl.pallas_call(
        _tc4_body,
        out_shape=[
            jax.ShapeDtypeStruct((NP, F), jnp.float32),
            jax.ShapeDtypeStruct((NP, F), jnp.float32),
        ],
    )(a1, sp, dis_col, w20, b2)


def _tc5_body(c0_ref, t_ref, dis_ref, w_ref, tx_out, c01_out):
    tx = dis_ref[...] * (t_ref[0] + t_ref[1])
    tx_out[...] = tx
    c01_out[...] = c0_ref[...] + jnp.dot(tx, w_ref[...],
                                         preferred_element_type=jnp.float32)


def _tc5(c0, t, dis_col, w):
    fo = w.shape[-1]
    return pl.pallas_call(
        _tc5_body,
        out_shape=[
            jax.ShapeDtypeStruct((NP, F), jnp.float32),
            jax.ShapeDtypeStruct((NP, fo), jnp.float32),
        ],
    )(c0, t, dis_col, w)


def _tc6_body(c01_ref, t_ref, dis_ref, h_ref, w22_ref, w30_ref, b3_ref,
              h2_out, c0b_out):
    tx2 = (2.0 * dis_ref[...]) * (t_ref[0] + t_ref[1]) - h_ref[...]
    h2 = jnp.maximum(
        c01_ref[...] + jnp.dot(tx2, w22_ref[...],
                               preferred_element_type=jnp.float32), 0.0)
    h2_out[...] = h2
    c0b_out[...] = jnp.dot(h2, w30_ref[...],
                           preferred_element_type=jnp.float32) + b3_ref[...]


def _tc6(c01, t2, dis_col, h, w22, w30, b3):
    return pl.pallas_call(
        _tc6_body,
        out_shape=[
            jax.ShapeDtypeStruct((NP, F), jnp.float32),
            jax.ShapeDtypeStruct((NP, 128), jnp.float32),
        ],
    )(c01, t2, dis_col, h, w22, w30, b3)


def _tc8_body(c01_ref, t_ref, dis_ref, h2_ref, w32_ref, batch_ref, wl_ref,
              bl_ref, out_ref):
    tx2 = (2.0 * dis_ref[...]) * (t_ref[0] + t_ref[1]) - h2_ref[...]
    h3 = jnp.maximum(
        c01_ref[...] + jnp.dot(tx2, w32_ref[...],
                               preferred_element_type=jnp.float32), 0.0)
    gids = lax.broadcasted_iota(jnp.int32, (1, G), 1)
    oh = (batch_ref[...] == gids).astype(jnp.float32)
    seg = lax.dot_general(oh, h3, (((0,), (0,)), ((), ())),
                          preferred_element_type=jnp.float32)
    cnt = jnp.sum(oh, axis=0)
    pooled = seg / jnp.maximum(cnt, 1.0)[:, None]
    out_ref[...] = jnp.dot(pooled, wl_ref[...],
                           preferred_element_type=jnp.float32) + bl_ref[...]


def _tc8(c01b, u2, dis_col, h2, w32, batchp, wl, bl):
    return pl.pallas_call(
        _tc8_body,
        out_shape=jax.ShapeDtypeStruct((G, wl.shape[-1]), jnp.float32),
    )(c01b, u2, dis_col, h2, w32, batchp, wl, bl)


def kernel(x, edge_index, edge_attr, batch, W1, b1, W2, b2, W3, b3, Wl, bl):
    src = edge_index[0].astype(jnp.int32)
    dst = edge_index[1].astype(jnp.int32)
    w = edge_attr.astype(jnp.float32)
    xp = jnp.pad(x, ((0, NP - N), (0, 0)))
    batchp = jnp.pad(batch.astype(jnp.int32), (0, NP - N),
                     constant_values=G).reshape(NP, 1)
    zeros1 = jnp.zeros((NP,), jnp.float32)
    zeros2 = jnp.zeros((NP, F), jnp.float32)
    w1cat = jnp.concatenate([W1[0], W1[1], W1[2]], axis=1)

    dis, norm = _prep_kernel(src, dst, w, zeros1)
    dis_col = dis.reshape(NP, 1)
    a1, y1, y2 = _tc1(xp, w1cat, b1.reshape(1, F))

    u = _spmm_next(y2, src, dst, norm, zeros2)
    v = _tc3(y1, u, dis_col)
    sp = _spmm_next(v, src, dst, norm, zeros2)
    h, c0 = _tc4(a1, sp, dis_col, W2[0], b2.reshape(1, F))

    t1 = _spmm_next(h, src, dst, norm, zeros2)
    tx1, c01 = _tc5(c0, t1, dis_col, W2[1])
    t2 = _spmm_next(tx1, src, dst, norm, zeros2)
    h2, c0b = _tc6(c01, t2, dis_col, h, W2[2], W3[0], b3.reshape(1, 128))

    u1 = _spmm_next(h2, src, dst, norm, zeros2)
    u1s, c01b = _tc5(c0b, u1, dis_col, W3[1])
    u2 = _spmm_next(u1s, src, dst, norm, zeros2)
    return _tc8(c01b, u2, dis_col, h2, W3[2], batchp, Wl,
                bl.reshape(1, Wl.shape[-1]))


# speedup vs baseline: 17.9115x; 1.1783x over previous
import dataclasses
import functools

import jax
import jax.numpy as jnp
from jax import lax
from jax.experimental import pallas as pl
from jax.experimental.pallas import tpu as pltpu
from jax.experimental.pallas import tpu_sc as plsc

N = 10000
NP = 10240
E = 320000
G = 8
NC = 2
NS = 16
NW = NC * NS
EW = E // NW
B = 80
NB = EW // B
RPS = NP // NS
F = 64

_mesh = plsc.VectorSubcoreMesh(core_axis_name="c", subcore_axis_name="s")

_sc_params = pltpu.CompilerParams()
if "needs_layout_passes" in pltpu.CompilerParams.__dataclass_fields__:
    _sc_params = dataclasses.replace(_sc_params, needs_layout_passes=False)
if "use_tc_tiling_on_sc" in pltpu.CompilerParams.__dataclass_fields__:
    _sc_params = dataclasses.replace(_sc_params, use_tc_tiling_on_sc=False)


EPS = E // NS
B2 = 2000
NB2 = EPS // B2


def _prep_body(src_hbm, dst_hbm, w_hbm, zeros_hbm, dis_out, norm_out,
               sbig, dbig, wbig, si, nv, dv, disv, acc, dis_sh):
    c = lax.axis_index("c")
    s = lax.axis_index("s")
    wid = c * NS + s
    rowbase = s * RPS
    ebase = s * EPS
    pltpu.sync_copy(zeros_hbm.at[pl.ds(rowbase, RPS)],
                    acc.at[pl.ds(rowbase, RPS)])
    pltpu.sync_copy(src_hbm.at[pl.ds(ebase, EPS)], sbig)
    pltpu.sync_copy(dst_hbm.at[pl.ds(ebase, EPS)], dbig)
    pltpu.sync_copy(w_hbm.at[pl.ds(ebase, EPS)], wbig)
    plsc.subcore_barrier()

    @pl.loop(0, EPS // 16, unroll=8)
    def _(g):
        sl = pl.ds(g * 16, 16)
        wbig[sl] = jnp.where(sbig[sl] != dbig[sl], wbig[sl], 0.0)

    @pl.loop(0, NB2)
    def _(blk):
        e0 = blk * B2
        pltpu.sync_copy(src_hbm.at[pl.ds(ebase + e0, B2)], si)
        pltpu.sync_copy(wbig.at[pl.ds(e0, B2)], acc.at[si], add=True)

    plsc.subcore_barrier()
    pltpu.sync_copy(acc.at[pl.ds(rowbase, RPS)], dv)

    @pl.loop(0, RPS // 16, unroll=4)
    def _(g):
        sl = pl.ds(g * 16, 16)
        x = dv[sl]
        i = plsc.bitcast(x, jnp.int32)
        y = plsc.bitcast(jnp.int32(0x5F3759DF) - lax.shift_right_logical(
            i, 1), jnp.float32)
        for _ in range(4):
            y = y * (1.5 - 0.5 * x * y * y)
        dv[sl] = jnp.where(x > 0, y, 0.0)

    hw = RPS // NC
    pltpu.sync_copy(dv.at[pl.ds(c * hw, hw)],
                    dis_out.at[pl.ds(rowbase + c * hw, hw)])
    pltpu.sync_copy(dv, dis_sh.at[pl.ds(rowbase, RPS)])
    plsc.subcore_barrier()
    pltpu.sync_copy(dis_sh, disv)

    wbase = wid * EW
    pltpu.sync_copy(src_hbm.at[pl.ds(wbase, EW)], sbig.at[pl.ds(0, EW)])
    pltpu.sync_copy(dst_hbm.at[pl.ds(wbase, EW)], dbig.at[pl.ds(0, EW)])
    pltpu.sync_copy(w_hbm.at[pl.ds(wbase, EW)], wbig.at[pl.ds(0, EW)])

    @pl.loop(0, EW // 16, unroll=8)
    def _(g):
        sl = pl.ds(g * 16, 16)
        wm = jnp.where(sbig[sl] != dbig[sl], wbig[sl], 0.0)
        nv[sl] = -wm * plsc.load_gather(disv, [sbig[sl]])

    pltpu.sync_copy(nv, norm_out.at[pl.ds(wbase, EW)])


def _prep_kernel(src, dst, w, zeros1):
    return pl.kernel(
        _prep_body,
        out_type=[jax.ShapeDtypeStruct((NP,), jnp.float32),
                  jax.ShapeDtypeStruct((E,), jnp.float32)],
        mesh=_mesh,
        scratch_types=[
            pltpu.VMEM((EPS,), jnp.int32),
            pltpu.VMEM((EPS,), jnp.int32),
            pltpu.VMEM((EPS,), jnp.float32),
            pltpu.VMEM((B2,), jnp.int32),
            pltpu.VMEM((EW,), jnp.float32),
            pltpu.VMEM((RPS,), jnp.float32),
            pltpu.VMEM((NP,), jnp.float32),
            pltpu.VMEM_SHARED((NP,), jnp.float32),
            pltpu.VMEM_SHARED((NP,), jnp.float32),
        ],
        compiler_params=_sc_params,
    )(src, dst, w, zeros1)


def _spmm_body(z_hbm, src_hbm, dst_hbm, nrm_hbm, zeros_hbm, out,
               sbig, nbig, di0, di1, di2, rows0, rows1, rows2, acc,
               gsem0, gsem1, gsem2, dsem0, dsem1, dsem2,
               ssem0, ssem1, ssem2):
    di = (di0, di1, di2)
    rows = (rows0, rows1, rows2)
    gsem = (gsem0, gsem1, gsem2)
    dsem = (dsem0, dsem1, dsem2)
    ssem = (ssem0, ssem1, ssem2)
    c = lax.axis_index("c")
    s = lax.axis_index("s")
    wid = c * NS + s
    rowbase = s * RPS
    ebase = wid * EW
    pltpu.sync_copy(zeros_hbm.at[pl.ds(rowbase, RPS)],
                    acc.at[pl.ds(rowbase, RPS)])
    pltpu.sync_copy(src_hbm.at[pl.ds(ebase, EW)], sbig)
    pltpu.sync_copy(nrm_hbm.at[pl.ds(ebase, EW)], nbig)
    plsc.subcore_barrier()

    def issue_in(k, p):
        pltpu.async_copy(dst_hbm.at[pl.ds(ebase + k * B, B)], di[p],
                         dsem[p])
        pltpu.async_copy(z_hbm.at[sbig.at[pl.ds(k * B, B)]], rows[p],
                         gsem[p])

    def wait_in(p):
        pltpu.make_async_copy(dst_hbm.at[pl.ds(ebase, B)], di[p],
                              dsem[p]).wait()
        pltpu.make_async_copy(z_hbm.at[sbig.at[pl.ds(0, B)]], rows[p],
                              gsem[p]).wait()

    def issue_scat(p):
        pltpu.async_copy(rows[p], acc.at[di[p]], ssem[p], add=True)

    def wait_scat(p):
        pltpu.make_async_copy(rows[p], acc.at[di[p]], ssem[p]).wait()

    def scale(k, p):
        e0 = k * B

        @plsc.parallel_loop(0, B, unroll=8)
        def _(i):
            n16 = plsc.load_gather(
                nbig, [jnp.zeros((16,), jnp.int32) + (e0 + i)])
            row = rows[p].at[i]
            for ch in range(F // 16):
                cs = pl.ds(ch * 16, 16)
                row[cs] = row[cs] * n16

    def body(k, p):
        wait_in(p)
        scale(k, p)
        issue_scat(p)
        q = (p + 2) % 3
        wait_scat(q)
        issue_in(k + 2, q)

    issue_in(0, 0)
    issue_in(1, 1)
    wait_in(0)
    scale(0, 0)
    issue_scat(0)
    issue_in(2, 2)

    nfull = NB - 3
    nloop = nfull // 3

    @pl.loop(0, nloop)
    def _(t):
        k = 3 * t + 1
        body(k, 1)
        body(k + 1, 2)
        body(k + 2, 0)

    for j in range(nfull % 3):
        kj = 3 * nloop + 1 + j
        body(kj, kj % 3)
    for kj in (NB - 2, NB - 1):
        wait_in(kj % 3)
        scale(kj, kj % 3)
        issue_scat(kj % 3)
    wait_scat((NB - 3) % 3)
    wait_scat((NB - 2) % 3)
    wait_scat((NB - 1) % 3)

    plsc.subcore_barrier()
    pltpu.sync_copy(acc.at[pl.ds(rowbase, RPS)],
                    out.at[c, pl.ds(rowbase, RPS)])


_spmm_next = pl.kernel(
    _spmm_body,
    out_type=jax.ShapeDtypeStruct((NC, NP, F), jnp.float32),
    mesh=_mesh,
    scratch_types=[
        pltpu.VMEM((EW,), jnp.int32),
        pltpu.VMEM((EW,), jnp.float32),
        pltpu.VMEM((B,), jnp.int32),
        pltpu.VMEM((B,), jnp.int32),
        pltpu.VMEM((B,), jnp.int32),
        pltpu.VMEM((B, F), jnp.float32),
        pltpu.VMEM((B, F), jnp.float32),
        pltpu.VMEM((B, F), jnp.float32),
        pltpu.VMEM_SHARED((NP, F), jnp.float32),
    ] + [pltpu.SemaphoreType.DMA] * 9,
    compiler_params=_sc_params,
)


def _tc1_body(x_ref, w_ref, b_ref, a1_out, y1_out, y2_out):
    y = jnp.dot(x_ref[...], w_ref[...], preferred_element_type=jnp.float32)
    y2 = y[:, 2 * F:]
    a1_out[...] = y[:, :F] - y2 + b_ref[...]
    y1_out[...] = y[:, F:2 * F]
    y2_out[...] = y2


def _tc1(xp, w1cat, b1):
    return pl.pallas_call(
        _tc1_body,
        out_shape=[
            jax.ShapeDtypeStruct((NP, F), jnp.float32),
            jax.ShapeDtypeStruct((NP, F), jnp.float32),
            jax.ShapeDtypeStruct((NP, F), jnp.float32),
        ],
    )(xp, w1cat, b1)


def _tc3_body(y1_ref, u_ref, dis_ref, v_out):
    v_out[...] = y1_ref[...] + (2.0 * dis_ref[...]) * (u_ref[0] + u_ref[1])


def _tc3(y1, u, dis_col):
    return pl.pallas_call(
        _tc3_body,
        out_shape=jax.ShapeDtypeStruct((NP, F), jnp.float32),
    )(y1, u, dis_col)


def _tc4_body(a1_ref, sp_ref, dis_ref, w_ref, b_ref, h_out, c0_out):
    h = jnp.maximum(a1_ref[...] + dis_ref[...] * (sp_ref[0] + sp_ref[1]), 0.0)
    h_out[...] = h
    c0_out[...] = jnp.dot(h, w_ref[...],
                          preferred_element_type=jnp.float32) + b_ref[...]


def _tc4(a1, sp, dis_col, w20, b2):
    return pl.pallas_call(
        _tc4_body,
        out_shape=[
            jax.ShapeDtypeStruct((NP, F), jnp.float32),
            jax.ShapeDtypeStruct((NP, F), jnp.float32),
        ],
    )(a1, sp, dis_col, w20, b2)


def _tc5_body(c0_ref, t_ref, dis_ref, w_ref, tx_out, c01_out):
    tx = dis_ref[...] * (t_ref[0] + t_ref[1])
    tx_out[...] = tx
    c01_out[...] = c0_ref[...] + jnp.dot(tx, w_ref[...],
                                         preferred_element_type=jnp.float32)


def _tc5(c0, t, dis_col, w):
    fo = w.shape[-1]
    return pl.pallas_call(
        _tc5_body,
        out_shape=[
            jax.ShapeDtypeStruct((NP, F), jnp.float32),
            jax.ShapeDtypeStruct((NP, fo), jnp.float32),
        ],
    )(c0, t, dis_col, w)


def _tc6_body(c01_ref, t_ref, dis_ref, h_ref, w22_ref, w30_ref, b3_ref,
              h2_out, c0b_out):
    tx2 = (2.0 * dis_ref[...]) * (t_ref[0] + t_ref[1]) - h_ref[...]
    h2 = jnp.maximum(
        c01_ref[...] + jnp.dot(tx2, w22_ref[...],
                               preferred_element_type=jnp.float32), 0.0)
    h2_out[...] = h2
    c0b_out[...] = jnp.dot(h2, w30_ref[...],
                           preferred_element_type=jnp.float32) + b3_ref[...]


def _tc6(c01, t2, dis_col, h, w22, w30, b3):
    return pl.pallas_call(
        _tc6_body,
        out_shape=[
            jax.ShapeDtypeStruct((NP, F), jnp.float32),
            jax.ShapeDtypeStruct((NP, 128), jnp.float32),
        ],
    )(c01, t2, dis_col, h, w22, w30, b3)


def _tc8_body(c01_ref, t_ref, dis_ref, h2_ref, w32_ref, batch_ref, wl_ref,
              bl_ref, out_ref):
    tx2 = (2.0 * dis_ref[...]) * (t_ref[0] + t_ref[1]) - h2_ref[...]
    h3 = jnp.maximum(
        c01_ref[...] + jnp.dot(tx2, w32_ref[...],
                               preferred_element_type=jnp.float32), 0.0)
    gids = lax.broadcasted_iota(jnp.int32, (1, G), 1)
    oh = (batch_ref[...] == gids).astype(jnp.float32)
    seg = lax.dot_general(oh, h3, (((0,), (0,)), ((), ())),
                          preferred_element_type=jnp.float32)
    cnt = jnp.sum(oh, axis=0)
    pooled = seg / jnp.maximum(cnt, 1.0)[:, None]
    out_ref[...] = jnp.dot(pooled, wl_ref[...],
                           preferred_element_type=jnp.float32) + bl_ref[...]


def _tc8(c01b, u2, dis_col, h2, w32, batchp, wl, bl):
    return pl.pallas_call(
        _tc8_body,
        out_shape=jax.ShapeDtypeStruct((G, wl.shape[-1]), jnp.float32),
    )(c01b, u2, dis_col, h2, w32, batchp, wl, bl)


def kernel(x, edge_index, edge_attr, batch, W1, b1, W2, b2, W3, b3, Wl, bl):
    src = edge_index[0].astype(jnp.int32)
    dst = edge_index[1].astype(jnp.int32)
    w = edge_attr.astype(jnp.float32)
    xp = jnp.pad(x, ((0, NP - N), (0, 0)))
    batchp = jnp.pad(batch.astype(jnp.int32), (0, NP - N),
                     constant_values=G).reshape(NP, 1)
    zeros1 = jnp.zeros((NP,), jnp.float32)
    zeros2 = jnp.zeros((NP, F), jnp.float32)
    w1cat = jnp.concatenate([W1[0], W1[1], W1[2]], axis=1)

    dis, norm = _prep_kernel(src, dst, w, zeros1)
    dis_col = dis.reshape(NP, 1)
    a1, y1, y2 = _tc1(xp, w1cat, b1.reshape(1, F))

    u = _spmm_next(y2, src, dst, norm, zeros2)
    v = _tc3(y1, u, dis_col)
    sp = _spmm_next(v, src, dst, norm, zeros2)
    h, c0 = _tc4(a1, sp, dis_col, W2[0], b2.reshape(1, F))

    t1 = _spmm_next(h, src, dst, norm, zeros2)
    tx1, c01 = _tc5(c0, t1, dis_col, W2[1])
    t2 = _spmm_next(tx1, src, dst, norm, zeros2)
    h2, c0b = _tc6(c01, t2, dis_col, h, W2[2], W3[0], b3.reshape(1, 128))

    u1 = _spmm_next(h2, src, dst, norm, zeros2)
    u1s, c01b = _tc5(c0b, u1, dis_col, W3[1])
    u2 = _spmm_next(u1s, src, dst, norm, zeros2)
    return _tc8(c01b, u2, dis_col, h2, W3[2], batchp, Wl,
                bl.reshape(1, Wl.shape[-1]))


# speedup vs baseline: 19.1775x; 1.0707x over previous
import dataclasses
import functools

import jax
import jax.numpy as jnp
from jax import lax
from jax.experimental import pallas as pl
from jax.experimental.pallas import tpu as pltpu
from jax.experimental.pallas import tpu_sc as plsc

N = 10000
NP = 10240
E = 320000
G = 8
NC = 2
NS = 16
NW = NC * NS
EW = E // NW
B = 80
NB = EW // B
RPS = NP // NS
F = 64

_mesh = plsc.VectorSubcoreMesh(core_axis_name="c", subcore_axis_name="s")

_sc_params = pltpu.CompilerParams()
if "needs_layout_passes" in pltpu.CompilerParams.__dataclass_fields__:
    _sc_params = dataclasses.replace(_sc_params, needs_layout_passes=False)
if "use_tc_tiling_on_sc" in pltpu.CompilerParams.__dataclass_fields__:
    _sc_params = dataclasses.replace(_sc_params, use_tc_tiling_on_sc=False)


EPS = E // NS
B2 = 2000
NB2 = EPS // B2


def _prep_body(src_hbm, dst_hbm, w_hbm, zeros_hbm, dis_out, norm_out,
               sbig, dbig, wbig, si, nv, dv, disv, acc, dis_sh):
    c = lax.axis_index("c")
    s = lax.axis_index("s")
    wid = c * NS + s
    rowbase = s * RPS
    ebase = s * EPS
    pltpu.sync_copy(zeros_hbm.at[pl.ds(rowbase, RPS)],
                    acc.at[pl.ds(rowbase, RPS)])
    pltpu.sync_copy(src_hbm.at[pl.ds(ebase, EPS)], sbig)
    pltpu.sync_copy(dst_hbm.at[pl.ds(ebase, EPS)], dbig)
    pltpu.sync_copy(w_hbm.at[pl.ds(ebase, EPS)], wbig)
    plsc.subcore_barrier()

    @pl.loop(0, EPS // 16, unroll=8)
    def _(g):
        sl = pl.ds(g * 16, 16)
        wbig[sl] = jnp.where(sbig[sl] != dbig[sl], wbig[sl], 0.0)

    @pl.loop(0, NB2)
    def _(blk):
        e0 = blk * B2
        pltpu.sync_copy(src_hbm.at[pl.ds(ebase + e0, B2)], si)
        pltpu.sync_copy(wbig.at[pl.ds(e0, B2)], acc.at[si], add=True)

    plsc.subcore_barrier()
    pltpu.sync_copy(acc.at[pl.ds(rowbase, RPS)], dv)

    @pl.loop(0, RPS // 16, unroll=4)
    def _(g):
        sl = pl.ds(g * 16, 16)
        x = dv[sl]
        i = plsc.bitcast(x, jnp.int32)
        y = plsc.bitcast(jnp.int32(0x5F3759DF) - lax.shift_right_logical(
            i, 1), jnp.float32)
        for _ in range(4):
            y = y * (1.5 - 0.5 * x * y * y)
        dv[sl] = jnp.where(x > 0, y, 0.0)

    hw = RPS // NC
    pltpu.sync_copy(dv.at[pl.ds(c * hw, hw)],
                    dis_out.at[pl.ds(rowbase + c * hw, hw)])
    pltpu.sync_copy(dv, dis_sh.at[pl.ds(rowbase, RPS)])
    plsc.subcore_barrier()
    pltpu.sync_copy(dis_sh, disv)

    wbase = wid * EW
    pltpu.sync_copy(src_hbm.at[pl.ds(wbase, EW)], sbig.at[pl.ds(0, EW)])
    pltpu.sync_copy(dst_hbm.at[pl.ds(wbase, EW)], dbig.at[pl.ds(0, EW)])
    pltpu.sync_copy(w_hbm.at[pl.ds(wbase, EW)], wbig.at[pl.ds(0, EW)])

    @pl.loop(0, EW // 16, unroll=8)
    def _(g):
        sl = pl.ds(g * 16, 16)
        wm = jnp.where(sbig[sl] != dbig[sl], wbig[sl], 0.0)
        nv[sl] = -wm * plsc.load_gather(disv, [sbig[sl]])

    pltpu.sync_copy(nv, norm_out.at[pl.ds(wbase, EW)])


def _prep_kernel(src, dst, w, zeros1):
    return pl.kernel(
        _prep_body,
        out_type=[jax.ShapeDtypeStruct((NP,), jnp.float32),
                  jax.ShapeDtypeStruct((E,), jnp.float32)],
        mesh=_mesh,
        scratch_types=[
            pltpu.VMEM((EPS,), jnp.int32),
            pltpu.VMEM((EPS,), jnp.int32),
            pltpu.VMEM((EPS,), jnp.float32),
            pltpu.VMEM((B2,), jnp.int32),
            pltpu.VMEM((EW,), jnp.float32),
            pltpu.VMEM((RPS,), jnp.float32),
            pltpu.VMEM((NP,), jnp.float32),
            pltpu.VMEM_SHARED((NP,), jnp.float32),
            pltpu.VMEM_SHARED((NP,), jnp.float32),
        ],
        compiler_params=_sc_params,
    )(src, dst, w, zeros1)


def _spmm_body(z_hbm, src_hbm, dst_hbm, nrm_hbm, zeros_hbm, out,
               sbig, nbig, di0, di1, di2, rows0, rows1, rows2,
               rb0, rb1, rb2, acc,
               gsem0, gsem1, gsem2, dsem0, dsem1, dsem2,
               ssem0, ssem1, ssem2):
    di = (di0, di1, di2)
    rows = (rows0, rows1, rows2)
    rowsb = (rb0, rb1, rb2)
    gsem = (gsem0, gsem1, gsem2)
    dsem = (dsem0, dsem1, dsem2)
    ssem = (ssem0, ssem1, ssem2)
    c = lax.axis_index("c")
    s = lax.axis_index("s")
    wid = c * NS + s
    rowbase = s * RPS
    ebase = wid * EW
    pltpu.sync_copy(zeros_hbm.at[pl.ds(rowbase, RPS)],
                    acc.at[pl.ds(rowbase, RPS)])
    pltpu.sync_copy(src_hbm.at[pl.ds(ebase, EW)], sbig)
    pltpu.sync_copy(nrm_hbm.at[pl.ds(ebase, EW)], nbig)
    plsc.subcore_barrier()

    def issue_in(k, p):
        pltpu.async_copy(dst_hbm.at[pl.ds(ebase + k * B, B)], di[p],
                         dsem[p])
        pltpu.async_copy(z_hbm.at[sbig.at[pl.ds(k * B, B)]], rowsb[p],
                         gsem[p])

    def wait_in(p):
        pltpu.make_async_copy(dst_hbm.at[pl.ds(ebase, B)], di[p],
                              dsem[p]).wait()
        pltpu.make_async_copy(z_hbm.at[sbig.at[pl.ds(0, B)]], rowsb[p],
                              gsem[p]).wait()

    def issue_scat(p):
        pltpu.async_copy(rows[p], acc.at[di[p]], ssem[p], add=True)

    def wait_scat(p):
        pltpu.make_async_copy(rows[p], acc.at[di[p]], ssem[p]).wait()

    def scale(k, p):
        e0 = k * B

        @plsc.parallel_loop(0, B, unroll=8)
        def _(i):
            n16 = plsc.load_gather(
                nbig, [jnp.zeros((16,), jnp.int32) + (e0 + i)])
            rowb = rowsb[p].at[i]
            rowf = rows[p].at[i]
            for ch in range(F // 32):
                a, b = plsc.unpack(rowb[pl.ds(ch * 32, 32)],
                                   format=plsc.PackFormat.INTERLEAVED)
                rowf[pl.ds(ch * 32, 16)] = a * n16
                rowf[pl.ds(ch * 32 + 16, 16)] = b * n16

    def body(k, p):
        wait_in(p)
        scale(k, p)
        issue_scat(p)
        q = (p + 2) % 3
        wait_scat(q)
        issue_in(k + 2, q)

    issue_in(0, 0)
    issue_in(1, 1)
    wait_in(0)
    scale(0, 0)
    issue_scat(0)
    issue_in(2, 2)

    nfull = NB - 3
    nloop = nfull // 3

    @pl.loop(0, nloop)
    def _(t):
        k = 3 * t + 1
        body(k, 1)
        body(k + 1, 2)
        body(k + 2, 0)

    for j in range(nfull % 3):
        kj = 3 * nloop + 1 + j
        body(kj, kj % 3)
    for kj in (NB - 2, NB - 1):
        wait_in(kj % 3)
        scale(kj, kj % 3)
        issue_scat(kj % 3)
    wait_scat((NB - 3) % 3)
    wait_scat((NB - 2) % 3)
    wait_scat((NB - 1) % 3)

    plsc.subcore_barrier()
    pltpu.sync_copy(acc.at[pl.ds(rowbase, RPS)],
                    out.at[c, pl.ds(rowbase, RPS)])


_spmm_next = pl.kernel(
    _spmm_body,
    out_type=jax.ShapeDtypeStruct((NC, NP, F), jnp.float32),
    mesh=_mesh,
    scratch_types=[
        pltpu.VMEM((EW,), jnp.int32),
        pltpu.VMEM((EW,), jnp.float32),
        pltpu.VMEM((B,), jnp.int32),
        pltpu.VMEM((B,), jnp.int32),
        pltpu.VMEM((B,), jnp.int32),
        pltpu.VMEM((B, F), jnp.float32),
        pltpu.VMEM((B, F), jnp.float32),
        pltpu.VMEM((B, F), jnp.float32),
        pltpu.VMEM((B, F), jnp.bfloat16),
        pltpu.VMEM((B, F), jnp.bfloat16),
        pltpu.VMEM((B, F), jnp.bfloat16),
        pltpu.VMEM_SHARED((NP, F), jnp.float32),
    ] + [pltpu.SemaphoreType.DMA] * 9,
    compiler_params=_sc_params,
)


def _tc1_body(x_ref, w_ref, b_ref, pm_ref, a1_out, y1_out, y2bf_out):
    y = jnp.dot(x_ref[...], w_ref[...], preferred_element_type=jnp.float32)
    y2 = y[:, 2 * F:]
    a1_out[...] = y[:, :F] - y2 + b_ref[...]
    y1_out[...] = y[:, F:2 * F]
    y2bf_out[...] = jnp.dot(y2, pm_ref[...],
                            preferred_element_type=jnp.float32
                            ).astype(jnp.bfloat16)


def _tc1(xp, w1cat, b1, pm):
    return pl.pallas_call(
        _tc1_body,
        out_shape=[
            jax.ShapeDtypeStruct((NP, F), jnp.float32),
            jax.ShapeDtypeStruct((NP, F), jnp.float32),
            jax.ShapeDtypeStruct((NP, F), jnp.bfloat16),
        ],
    )(xp, w1cat, b1, pm)


def _tc3_body(y1_ref, u_ref, dis_ref, pm_ref, vbf_out):
    v = y1_ref[...] + (2.0 * dis_ref[...]) * (u_ref[0] + u_ref[1])
    vbf_out[...] = jnp.dot(v, pm_ref[...],
                           preferred_element_type=jnp.float32
                           ).astype(jnp.bfloat16)


def _tc3(y1, u, dis_col, pm):
    return pl.pallas_call(
        _tc3_body,
        out_shape=jax.ShapeDtypeStruct((NP, F), jnp.bfloat16),
    )(y1, u, dis_col, pm)


def _tc4_body(a1_ref, sp_ref, dis_ref, w_ref, b_ref, pm_ref,
              h_out, hbf_out, c0_out):
    h = jnp.maximum(a1_ref[...] + dis_ref[...] * (sp_ref[0] + sp_ref[1]), 0.0)
    h_out[...] = h
    hbf_out[...] = jnp.dot(h, pm_ref[...],
                           preferred_element_type=jnp.float32
                           ).astype(jnp.bfloat16)
    c0_out[...] = jnp.dot(h, w_ref[...],
                          preferred_element_type=jnp.float32) + b_ref[...]


def _tc4(a1, sp, dis_col, w20, b2, pm):
    return pl.pallas_call(
        _tc4_body,
        out_shape=[
            jax.ShapeDtypeStruct((NP, F), jnp.float32),
            jax.ShapeDtypeStruct((NP, F), jnp.bfloat16),
            jax.ShapeDtypeStruct((NP, F), jnp.float32),
        ],
    )(a1, sp, dis_col, w20, b2, pm)


def _tc5_body(c0_ref, t_ref, dis_ref, w_ref, pm_ref, txbf_out, c01_out):
    tx = dis_ref[...] * (t_ref[0] + t_ref[1])
    txbf_out[...] = jnp.dot(tx, pm_ref[...],
                            preferred_element_type=jnp.float32
                            ).astype(jnp.bfloat16)
    c01_out[...] = c0_ref[...] + jnp.dot(tx, w_ref[...],
                                         preferred_element_type=jnp.float32)


def _tc5(c0, t, dis_col, w, pm):
    fo = w.shape[-1]
    return pl.pallas_call(
        _tc5_body,
        out_shape=[
            jax.ShapeDtypeStruct((NP, F), jnp.bfloat16),
            jax.ShapeDtypeStruct((NP, fo), jnp.float32),
        ],
    )(c0, t, dis_col, w, pm)


def _tc6_body(c01_ref, t_ref, dis_ref, h_ref, w22_ref, w30_ref, b3_ref,
              pm_ref, h2_out, h2bf_out, c0b_out):
    tx2 = (2.0 * dis_ref[...]) * (t_ref[0] + t_ref[1]) - h_ref[...]
    h2 = jnp.maximum(
        c01_ref[...] + jnp.dot(tx2, w22_ref[...],
                               preferred_element_type=jnp.float32), 0.0)
    h2_out[...] = h2
    h2bf_out[...] = jnp.dot(h2, pm_ref[...],
                            preferred_element_type=jnp.float32
                            ).astype(jnp.bfloat16)
    c0b_out[...] = jnp.dot(h2, w30_ref[...],
                           preferred_element_type=jnp.float32) + b3_ref[...]


def _tc6(c01, t2, dis_col, h, w22, w30, b3, pm):
    return pl.pallas_call(
        _tc6_body,
        out_shape=[
            jax.ShapeDtypeStruct((NP, F), jnp.float32),
            jax.ShapeDtypeStruct((NP, F), jnp.bfloat16),
            jax.ShapeDtypeStruct((NP, 128), jnp.float32),
        ],
    )(c01, t2, dis_col, h, w22, w30, b3, pm)


def _tc8_body(c01_ref, t_ref, dis_ref, h2_ref, w32_ref, batch_ref, wl_ref,
              bl_ref, out_ref):
    tx2 = (2.0 * dis_ref[...]) * (t_ref[0] + t_ref[1]) - h2_ref[...]
    h3 = jnp.maximum(
        c01_ref[...] + jnp.dot(tx2, w32_ref[...],
                               preferred_element_type=jnp.float32), 0.0)
    gids = lax.broadcasted_iota(jnp.int32, (1, G), 1)
    oh = (batch_ref[...] == gids).astype(jnp.float32)
    seg = lax.dot_general(oh, h3, (((0,), (0,)), ((), ())),
                          preferred_element_type=jnp.float32)
    cnt = jnp.sum(oh, axis=0)
    pooled = seg / jnp.maximum(cnt, 1.0)[:, None]
    out_ref[...] = jnp.dot(pooled, wl_ref[...],
                           preferred_element_type=jnp.float32) + bl_ref[...]


def _tc8(c01b, u2, dis_col, h2, w32, batchp, wl, bl):
    return pl.pallas_call(
        _tc8_body,
        out_shape=jax.ShapeDtypeStruct((G, wl.shape[-1]), jnp.float32),
    )(c01b, u2, dis_col, h2, w32, batchp, wl, bl)


def kernel(x, edge_index, edge_attr, batch, W1, b1, W2, b2, W3, b3, Wl, bl):
    src = edge_index[0].astype(jnp.int32)
    dst = edge_index[1].astype(jnp.int32)
    w = edge_attr.astype(jnp.float32)
    xp = jnp.pad(x, ((0, NP - N), (0, 0)))
    batchp = jnp.pad(batch.astype(jnp.int32), (0, NP - N),
                     constant_values=G).reshape(NP, 1)
    zeros1 = jnp.zeros((NP,), jnp.float32)
    zeros2 = jnp.zeros((NP, F), jnp.float32)
    w1cat = jnp.concatenate([W1[0], W1[1], W1[2]], axis=1)
    m = jnp.arange(F)
    q = 32 * (m // 32) + (m % 32) // 2 + 16 * (m % 2)
    pm = jax.nn.one_hot(q, F, dtype=jnp.float32)
    pm = pm.T

    dis, norm = _prep_kernel(src, dst, w, zeros1)
    dis_col = dis.reshape(NP, 1)
    a1, y1, y2bf = _tc1(xp, w1cat, b1.reshape(1, F), pm)

    u = _spmm_next(y2bf, src, dst, norm, zeros2)
    vbf = _tc3(y1, u, dis_col, pm)
    sp = _spmm_next(vbf, src, dst, norm, zeros2)
    h, hbf, c0 = _tc4(a1, sp, dis_col, W2[0], b2.reshape(1, F), pm)

    t1 = _spmm_next(hbf, src, dst, norm, zeros2)
    txbf, c01 = _tc5(c0, t1, dis_col, W2[1], pm)
    t2 = _spmm_next(txbf, src, dst, norm, zeros2)
    h2, h2bf, c0b = _tc6(c01, t2, dis_col, h, W2[2], W3[0],
                         b3.reshape(1, 128), pm)

    u1 = _spmm_next(h2bf, src, dst, norm, zeros2)
    u1sbf, c01b = _tc5(c0b, u1, dis_col, W3[1], pm)
    u2 = _spmm_next(u1sbf, src, dst, norm, zeros2)
    return _tc8(c01b, u2, dis_col, h2, W3[2], batchp, Wl,
                bl.reshape(1, Wl.shape[-1]))
